# baseline (speedup 1.0000x reference)
import functools

import jax
import jax.numpy as jnp
from jax import lax
from jax.experimental import pallas as pl
from jax.experimental.pallas import tpu as pltpu

T = 36
POOL = 3
CONV_KS = (10, 15)


def _sigmoid(x):
    return 0.5 * (jnp.tanh(0.5 * x) + 1.0)


def _full(shape):
    nd = len(shape)
    return pl.BlockSpec(tuple(shape), lambda d, _n=nd: (0,) * _n)


def _cell(gates, c_prev, H):
    i = _sigmoid(gates[:, 0 * H:1 * H])
    f = _sigmoid(gates[:, 1 * H:2 * H])
    g = jnp.tanh(gates[:, 2 * H:3 * H])
    o = _sigmoid(gates[:, 3 * H:4 * H])
    c = f * c_prev + i * g
    return o * jnp.tanh(c), c


def _l1_kernel(p0_ref, p1_ref, p2_ref, cw_ref, cb_ref,
               wih1_ref, b1_ref, whh1_ref,
               l1_ref, feat_scr, xp_scr, *, Bp, H):
    f32 = jnp.float32
    d = pl.program_id(0)

    cw = cw_ref[...]
    y = jnp.maximum(
        jnp.maximum(jnp.dot(p0_ref[...], cw, preferred_element_type=f32),
                    jnp.dot(p1_ref[...], cw, preferred_element_type=f32)),
        jnp.dot(p2_ref[...], cw, preferred_element_type=f32))
    feat_scr[...] = jnp.maximum(y + cb_ref[...], 0.0)

    xp_scr[...] = jnp.dot(feat_scr[...], wih1_ref[...],
                          preferred_element_type=f32) + b1_ref[...]
    whh = whh1_ref[0]

    z = jnp.zeros((Bp, H), f32)

    def body(s, carry):
        h, c = carry
        t = lax.select(d == 0, s, T - 1 - s)
        r = pl.multiple_of(t * Bp, Bp)
        g = xp_scr[pl.ds(r, Bp), :] + jnp.dot(h, whh, preferred_element_type=f32)
        h, c = _cell(g, c, H)
        l1_ref[0, pl.ds(r, Bp), :] = h
        return h, c

    lax.fori_loop(0, T, body, (z, z))


def _l2_kernel(l1_ref, wih2_ref, b2_ref, whh2_ref, fc1w_ref,
               acc_ref, xp_scr, *, Bp, H, FCH):
    f32 = jnp.float32
    d = pl.program_id(0)

    xp_scr[...] = (jnp.dot(l1_ref[0], wih2_ref[0], preferred_element_type=f32)
                   + jnp.dot(l1_ref[1], wih2_ref[1], preferred_element_type=f32)
                   + b2_ref[...])
    whh = whh2_ref[0]

    z = jnp.zeros((Bp, H), f32)

    def body(s, carry):
        h, c, acc = carry
        t = lax.select(d == 0, s, T - 1 - s)
        r = pl.multiple_of(t * Bp, Bp)
        g = xp_scr[pl.ds(r, Bp), :] + jnp.dot(h, whh, preferred_element_type=f32)
        h, c = _cell(g, c, H)
        w = fc1w_ref[0, pl.ds(pl.multiple_of(t * H, H), H), :]
        acc = acc + jnp.dot(h, w, preferred_element_type=f32)
        return h, c, acc

    carry = lax.fori_loop(0, T, body, (z, z, jnp.zeros((Bp, FCH), f32)))
    acc_ref[0] = carry[2]


def _head_kernel(acc_ref, fc1b_ref, fc2w_ref, fc2b_ref, fc3w_ref, fc3b_ref,
                 o_ref):
    f32 = jnp.float32
    y = jnp.maximum(acc_ref[0] + acc_ref[1] + fc1b_ref[...], 0.0)
    y = jnp.maximum(jnp.dot(y, fc2w_ref[...], preferred_element_type=f32)
                    + fc2b_ref[...], 0.0)
    o_ref[...] = jnp.sum(y * fc3w_ref[...], axis=1, keepdims=True) + fc3b_ref[...]


def kernel(x, cw, cb, wih1, b1, whh1f, whh1b, wih2f, wih2b, b2, whh2f, whh2b,
           fc1wf, fc1wb, fc1b, fc2w, fc2b, fc3w, fc3b):
    f32 = jnp.float32
    B, L, Cin = x.shape
    H = whh1f.shape[0]
    FCH = fc2w.shape[0]
    C = cw.shape[1]
    CK = cw.shape[0]
    Bp = max(8, (B + 7) // 8 * 8)

    xb = jnp.pad(x.astype(f32), ((0, Bp - B), (0, 0), (0, 0)))
    x_bcl = jnp.transpose(xb, (0, 2, 1))

    plist = []
    for K in CONV_KS:
        pad_l = (K - 1) // 2
        pad_r = (K - 1) - pad_l
        xpd = jnp.pad(x_bcl, ((0, 0), (0, 0), (pad_l, pad_r)))
        idx = (POOL * jnp.arange(T)[:, None, None]
               + jnp.arange(POOL)[None, :, None]
               + jnp.arange(K)[None, None, :])
        pt = xpd[:, :, idx]
        pt = jnp.transpose(pt, (3, 2, 0, 1, 4)).reshape(POOL, T * Bp, Cin * K)
        plist.append(pt)
    patches = jnp.concatenate(plist, axis=-1)

    whh1 = jnp.stack([whh1f, whh1b])
    whh2 = jnp.stack([whh2f, whh2b])
    wih2 = jnp.stack([wih2f, wih2b])
    fc1w = jnp.stack([fc1wf, fc1wb])

    l1 = pl.pallas_call(
        functools.partial(_l1_kernel, Bp=Bp, H=H),
        out_shape=jax.ShapeDtypeStruct((2, T * Bp, H), f32),
        grid=(2,),
        in_specs=[
            _full((T * Bp, CK)), _full((T * Bp, CK)), _full((T * Bp, CK)),
            _full((CK, C)), _full((1, C)),
            pl.BlockSpec((C, 4 * H), lambda d: (0, d)),
            pl.BlockSpec((1, 4 * H), lambda d: (0, d)),
            pl.BlockSpec((1, H, 4 * H), lambda d: (d, 0, 0)),
        ],
        out_specs=pl.BlockSpec((1, T * Bp, H), lambda d: (d, 0, 0)),
        scratch_shapes=[
            pltpu.VMEM((T * Bp, C), f32),
            pltpu.VMEM((T * Bp, 4 * H), f32),
        ],
        compiler_params=pltpu.CompilerParams(
            dimension_semantics=("parallel",)),
    )(patches[0], patches[1], patches[2], cw, cb, wih1, b1, whh1)

    acc = pl.pallas_call(
        functools.partial(_l2_kernel, Bp=Bp, H=H, FCH=FCH),
        out_shape=jax.ShapeDtypeStruct((2, Bp, FCH), f32),
        grid=(2,),
        in_specs=[
            _full((2, T * Bp, H)),
            pl.BlockSpec((2, H, 4 * H), lambda d: (0, 0, d)),
            pl.BlockSpec((1, 4 * H), lambda d: (0, d)),
            pl.BlockSpec((1, H, 4 * H), lambda d: (d, 0, 0)),
            pl.BlockSpec((1, T * H, FCH), lambda d: (d, 0, 0)),
        ],
        out_specs=pl.BlockSpec((1, Bp, FCH), lambda d: (d, 0, 0)),
        scratch_shapes=[
            pltpu.VMEM((T * Bp, 4 * H), f32),
        ],
        compiler_params=pltpu.CompilerParams(
            dimension_semantics=("parallel",)),
    )(l1, wih2, b2, whh2, fc1w)

    out = pl.pallas_call(
        _head_kernel,
        out_shape=jax.ShapeDtypeStruct((Bp, 1), f32),
        grid=(1,),
        in_specs=[
            _full((2, Bp, FCH)), _full((1, FCH)),
            _full((FCH, FCH)), _full((1, FCH)),
            _full((1, FCH)), _full((1, 1)),
        ],
        out_specs=_full((Bp, 1)),
        compiler_params=pltpu.CompilerParams(
            dimension_semantics=("arbitrary",)),
    )(acc, fc1b, fc2w, fc2b, fc3w, fc3b)

    return out[:B, 0]


# speedup vs baseline: 1.4138x; 1.4138x over previous
import functools

import jax
import jax.numpy as jnp
from jax import lax
from jax.experimental import pallas as pl
from jax.experimental.pallas import tpu as pltpu

T = 36
POOL = 3
CONV_KS = (10, 15)
MC = 144
bf16 = jnp.bfloat16


def _sigmoid(x):
    return 0.5 * (jnp.tanh(0.5 * x) + 1.0)


def _full(shape):
    nd = len(shape)
    return pl.BlockSpec(tuple(shape), lambda d, _n=nd: (0,) * _n)


def _mm576(lhs_ref, col0, mxu, lsr):
    for j, mc in enumerate(range(0, T * 16, MC)):
        chunk = lhs_ref[pl.ds(mc, MC), pl.ds(col0, 256)].astype(bf16)
        pltpu.matmul_acc_lhs(mc // 4, chunk, mxu,
                             load_staged_rhs=lsr if j == 0 else None)


def _pop576(out_ref, col0, mxu, bias):
    for mc in range(0, T * 16, MC):
        v = pltpu.matmul_pop(mc // 4, (MC, 256), jnp.float32, mxu)
        out_ref[pl.ds(mc, MC), pl.ds(col0, 256)] = v + bias


def _lstm_dir_loop(d, xp_scr, whh16_scr, store_h, Bp, H):
    f32 = jnp.float32
    z = jnp.zeros((Bp, H), f32)

    def body(s, carry):
        h, c = carry
        t = lax.select(d == 0, s, T - 1 - s)
        r = pl.multiple_of(t * Bp, Bp)
        h16 = h.astype(bf16)
        for mxu in range(2):
            pltpu.matmul_push_rhs(
                whh16_scr[:, pl.ds((2 * mxu) * 256, 256)], 0, mxu)
            pltpu.matmul_acc_lhs(0, h16, mxu, load_staged_rhs=0)
            pltpu.matmul_push_rhs(
                whh16_scr[:, pl.ds((2 * mxu + 1) * 256, 256)], 1, mxu)
            pltpu.matmul_acc_lhs(8, h16, mxu, load_staged_rhs=1)
        xp = xp_scr[pl.ds(r, Bp), :]
        gi = pltpu.matmul_pop(0, (Bp, 256), f32, 0) + xp[:, 0:256]
        gf = pltpu.matmul_pop(8, (Bp, 256), f32, 0) + xp[:, 256:512]
        gg = pltpu.matmul_pop(0, (Bp, 256), f32, 1) + xp[:, 512:768]
        go = pltpu.matmul_pop(8, (Bp, 256), f32, 1) + xp[:, 768:1024]
        i = _sigmoid(gi)
        f = _sigmoid(gf)
        g = jnp.tanh(gg)
        o = _sigmoid(go)
        c = f * c + i * g
        h = o * jnp.tanh(c)
        store_h(r, t, h)
        return h, c

    lax.fori_loop(0, T, body, (z, z))


def _l1_kernel(p0_ref, p1_ref, p2_ref, cw_ref, cb_ref,
               wih1_ref, b1_ref, whh1f_ref, whh1b_ref,
               l1_ref, pscr, cwscr, feat_scr, xp_scr, whh16_scr, *, Bp, H):
    f32 = jnp.float32
    d = pl.program_id(0)
    CK = cw_ref.shape[0]

    cwscr[...] = jnp.zeros((256, 256), f32)
    cwscr[pl.ds(0, CK), :] = cw_ref[...]
    pscr[...] = jnp.zeros((T * Bp, 256), f32)

    for p, p_ref in enumerate((p0_ref, p1_ref, p2_ref)):
        pscr[:, pl.ds(0, CK)] = p_ref[...]
        mxu = p % 2
        pltpu.matmul_push_rhs(cwscr[...].astype(bf16), 0, mxu)
        _mm576(pscr, 0, mxu, 0)
        for mc in range(0, T * Bp, MC):
            v = pltpu.matmul_pop(mc // 4, (MC, 256), f32, mxu)
            if p == 0:
                feat_scr[pl.ds(mc, MC), :] = v
            elif p == 1:
                feat_scr[pl.ds(mc, MC), :] = jnp.maximum(
                    feat_scr[pl.ds(mc, MC), :], v)
            else:
                feat_scr[pl.ds(mc, MC), :] = jnp.maximum(
                    jnp.maximum(feat_scr[pl.ds(mc, MC), :], v) + cb_ref[...],
                    0.0)

    for n in range(4):
        mxu = n % 2
        pltpu.matmul_push_rhs(
            wih1_ref[:, pl.ds(n * 256, 256)].astype(bf16), 0, mxu)
        _mm576(feat_scr, 0, mxu, 0)
        _pop576(xp_scr, n * 256, mxu, b1_ref[0, pl.ds(n * 256, 256)][None, :])

    whh16_scr[...] = jnp.where(d == 0, whh1f_ref[...],
                               whh1b_ref[...]).astype(bf16)

    def store_h(r, t, h):
        l1_ref[0, pl.ds(r, Bp), :] = h

    _lstm_dir_loop(d, xp_scr, whh16_scr, store_h, Bp, H)


def _l2_kernel(l1_ref, wih2f_ref, wih2b_ref, b2_ref, whh2f_ref, whh2b_ref,
               fc1wf_hbm, fc1wb_hbm,
               acc_ref, xp_scr, h2_scr, fc1w_scr, whh16_scr, sem,
               *, Bp, H, FCH):
    f32 = jnp.float32
    d = pl.program_id(0)

    @pl.when(d == 0)
    def _():
        pltpu.make_async_copy(fc1wf_hbm, fc1w_scr, sem).start()

    @pl.when(d != 0)
    def _():
        pltpu.make_async_copy(fc1wb_hbm, fc1w_scr, sem).start()

    for n in range(4):
        mxu = n % 2
        pltpu.matmul_push_rhs(
            wih2f_ref[:, pl.ds(n * 256, 256)].astype(bf16), 0, mxu)
        pltpu.matmul_push_rhs(
            wih2b_ref[:, pl.ds(n * 256, 256)].astype(bf16), 1, mxu)
        _mm576(l1_ref.at[0], 0, mxu, 0)
        _mm576(l1_ref.at[1], 0, mxu, 1)
        _pop576(xp_scr, n * 256, mxu, b2_ref[0, pl.ds(n * 256, 256)][None, :])

    whh16_scr[...] = jnp.where(d == 0, whh2f_ref[...],
                               whh2b_ref[...]).astype(bf16)

    def store_h(r, t, h):
        h2_scr[:, pl.ds(pl.multiple_of(t * H, H), H)] = h

    _lstm_dir_loop(d, xp_scr, whh16_scr, store_h, Bp, H)

    pltpu.make_async_copy(fc1wf_hbm, fc1w_scr, sem).wait()
    for kt in range(T):
        mxu = kt % 2
        msr = (kt // 2) % 2
        pltpu.matmul_push_rhs(
            fc1w_scr[pl.ds(kt * 256, 256), :].astype(bf16), msr, mxu)
        pltpu.matmul_acc_lhs(0, h2_scr[:, pl.ds(kt * 256, 256)].astype(bf16),
                             mxu, load_staged_rhs=msr)
    acc_ref[0] = (pltpu.matmul_pop(0, (Bp, FCH), f32, 0)
                  + pltpu.matmul_pop(0, (Bp, FCH), f32, 1))


def _head_kernel(acc_ref, fc1b_ref, fc2w_ref, fc2b_ref, fc3w_ref, fc3b_ref,
                 o_ref):
    f32 = jnp.float32
    y = jnp.maximum(acc_ref[0] + acc_ref[1] + fc1b_ref[...], 0.0)
    y = jnp.maximum(jnp.dot(y, fc2w_ref[...], preferred_element_type=f32)
                    + fc2b_ref[...], 0.0)
    o_ref[...] = jnp.sum(y * fc3w_ref[...], axis=1, keepdims=True) + fc3b_ref[...]


def kernel(x, cw, cb, wih1, b1, whh1f, whh1b, wih2f, wih2b, b2, whh2f, whh2b,
           fc1wf, fc1wb, fc1b, fc2w, fc2b, fc3w, fc3b):
    f32 = jnp.float32
    B, L, Cin = x.shape
    H = whh1f.shape[0]
    FCH = fc2w.shape[0]
    C = cw.shape[1]
    CK = cw.shape[0]
    Bp = max(8, (B + 7) // 8 * 8)

    xb = jnp.pad(x.astype(f32), ((0, Bp - B), (0, 0), (0, 0)))
    x_bcl = jnp.transpose(xb, (0, 2, 1))

    plist = []
    for K in CONV_KS:
        pad_l = (K - 1) // 2
        pad_r = (K - 1) - pad_l
        xpd = jnp.pad(x_bcl, ((0, 0), (0, 0), (pad_l, pad_r)))
        idx = (POOL * jnp.arange(T)[:, None, None]
               + jnp.arange(POOL)[None, :, None]
               + jnp.arange(K)[None, None, :])
        pt = xpd[:, :, idx]
        pt = jnp.transpose(pt, (3, 2, 0, 1, 4)).reshape(POOL, T * Bp, Cin * K)
        plist.append(pt)
    patches = jnp.concatenate(plist, axis=-1)

    l1 = pl.pallas_call(
        functools.partial(_l1_kernel, Bp=Bp, H=H),
        out_shape=jax.ShapeDtypeStruct((2, T * Bp, H), f32),
        grid=(2,),
        in_specs=[
            _full((T * Bp, CK)), _full((T * Bp, CK)), _full((T * Bp, CK)),
            _full((CK, C)), _full((1, C)),
            pl.BlockSpec((C, 4 * H), lambda d: (0, d)),
            pl.BlockSpec((1, 4 * H), lambda d: (0, d)),
            _full((H, 4 * H)), _full((H, 4 * H)),
        ],
        out_specs=pl.BlockSpec((1, T * Bp, H), lambda d: (d, 0, 0)),
        scratch_shapes=[
            pltpu.VMEM((T * Bp, 256), f32),
            pltpu.VMEM((256, 256), f32),
            pltpu.VMEM((T * Bp, C), f32),
            pltpu.VMEM((T * Bp, 4 * H), f32),
            pltpu.VMEM((H, 4 * H), bf16),
        ],
        compiler_params=pltpu.CompilerParams(
            dimension_semantics=("parallel",)),
    )(patches[0], patches[1], patches[2], cw, cb, wih1, b1, whh1f, whh1b)

    acc = pl.pallas_call(
        functools.partial(_l2_kernel, Bp=Bp, H=H, FCH=FCH),
        out_shape=jax.ShapeDtypeStruct((2, Bp, FCH), f32),
        grid=(2,),
        in_specs=[
            _full((2, T * Bp, H)),
            pl.BlockSpec((H, 4 * H), lambda d: (0, d)),
            pl.BlockSpec((H, 4 * H), lambda d: (0, d)),
            pl.BlockSpec((1, 4 * H), lambda d: (0, d)),
            _full((H, 4 * H)), _full((H, 4 * H)),
            pl.BlockSpec(memory_space=pl.ANY),
            pl.BlockSpec(memory_space=pl.ANY),
        ],
        out_specs=pl.BlockSpec((1, Bp, FCH), lambda d: (d, 0, 0)),
        scratch_shapes=[
            pltpu.VMEM((T * Bp, 4 * H), f32),
            pltpu.VMEM((Bp, T * H), f32),
            pltpu.VMEM((T * H, FCH), f32),
            pltpu.VMEM((H, 4 * H), bf16),
            pltpu.SemaphoreType.DMA,
        ],
        compiler_params=pltpu.CompilerParams(
            dimension_semantics=("parallel",)),
    )(l1, wih2f, wih2b, b2, whh2f, whh2b, fc1wf, fc1wb)

    out = pl.pallas_call(
        _head_kernel,
        out_shape=jax.ShapeDtypeStruct((Bp, 1), f32),
        grid=(1,),
        in_specs=[
            _full((2, Bp, FCH)), _full((1, FCH)),
            _full((FCH, FCH)), _full((1, FCH)),
            _full((1, FCH)), _full((1, 1)),
        ],
        out_specs=_full((Bp, 1)),
        compiler_params=pltpu.CompilerParams(
            dimension_semantics=("arbitrary",)),
    )(acc, fc1b, fc2w, fc2b, fc3w, fc3b)

    return out[:B, 0]


# speedup vs baseline: 1.6931x; 1.1975x over previous
import functools

import jax
import jax.numpy as jnp
from jax import lax
from jax.experimental import pallas as pl
from jax.experimental.pallas import tpu as pltpu

T = 36
POOL = 3
CONV_KS = (10, 15)
MC = 144
bf16 = jnp.bfloat16


def _sigmoid(x):
    return 0.5 * (jnp.tanh(0.5 * x) + 1.0)


def _full(shape):
    nd = len(shape)
    return pl.BlockSpec(tuple(shape), lambda d, _n=nd: (0,) * _n)


def _mm576(lhs_ref, col0, mxu, lsr):
    for j, mc in enumerate(range(0, T * 16, MC)):
        chunk = lhs_ref[pl.ds(mc, MC), pl.ds(col0, 256)].astype(bf16)
        pltpu.matmul_acc_lhs(mc // 4, chunk, mxu,
                             load_staged_rhs=lsr if j == 0 else None)


def _pop576(out_ref, col0, mxu, bias):
    for mc in range(0, T * 16, MC):
        v = pltpu.matmul_pop(mc // 4, (MC, 256), jnp.float32, mxu)
        out_ref[pl.ds(mc, MC), pl.ds(col0, 256)] = v + bias


def _lstm_dir_loop(d, xp_scr, whh16_scr, store_h, Bp, H):
    f32 = jnp.float32
    z = jnp.zeros((Bp, H), f32)

    def body(s, carry):
        h, c = carry
        t = lax.select(d == 0, s, T - 1 - s)
        r = pl.multiple_of(t * Bp, Bp)
        h16 = h.astype(bf16)
        for mxu in range(2):
            pltpu.matmul_push_rhs(
                whh16_scr[:, pl.ds((2 * mxu) * 256, 256)], 0, mxu)
            pltpu.matmul_acc_lhs(0, h16, mxu, load_staged_rhs=0)
            pltpu.matmul_push_rhs(
                whh16_scr[:, pl.ds((2 * mxu + 1) * 256, 256)], 1, mxu)
            pltpu.matmul_acc_lhs(8, h16, mxu, load_staged_rhs=1)
        xp = xp_scr[pl.ds(r, Bp), :]
        gi = pltpu.matmul_pop(0, (Bp, 256), f32, 0) + xp[:, 0:256]
        gf = pltpu.matmul_pop(8, (Bp, 256), f32, 0) + xp[:, 256:512]
        gg = pltpu.matmul_pop(0, (Bp, 256), f32, 1) + xp[:, 512:768]
        go = pltpu.matmul_pop(8, (Bp, 256), f32, 1) + xp[:, 768:1024]
        i = _sigmoid(gi)
        f = _sigmoid(gf)
        g = jnp.tanh(gg)
        o = _sigmoid(go)
        c = f * c + i * g
        h = o * jnp.tanh(c)
        store_h(r, t, h)
        return h, c

    lax.fori_loop(0, T, body, (z, z))


def _l1_kernel(sp_ref, cwp0_ref, cwp1_ref, cwp2_ref, cb_ref,
               wih1_ref, b1_ref, whh1f_ref, whh1b_ref,
               l1_ref, pscr, cwscr, feat_scr, xp_scr, whh16_scr, *, Bp, H):
    f32 = jnp.float32
    d = pl.program_id(0)
    CK2 = cwp0_ref.shape[0]

    pscr[...] = jnp.zeros((T * Bp, 256), f32)
    pscr[:, pl.ds(0, CK2)] = sp_ref[...]
    cwscr[...] = jnp.zeros((256, 256), f32)

    for p, w_ref in enumerate((cwp0_ref, cwp1_ref, cwp2_ref)):
        cwscr[pl.ds(0, CK2), :] = w_ref[...]
        mxu = p % 2
        pltpu.matmul_push_rhs(cwscr[...].astype(bf16), 0, mxu)
        _mm576(pscr, 0, mxu, 0)
        for mc in range(0, T * Bp, MC):
            v = pltpu.matmul_pop(mc // 4, (MC, 256), f32, mxu)
            if p == 0:
                feat_scr[pl.ds(mc, MC), :] = v
            elif p == 1:
                feat_scr[pl.ds(mc, MC), :] = jnp.maximum(
                    feat_scr[pl.ds(mc, MC), :], v)
            else:
                feat_scr[pl.ds(mc, MC), :] = jnp.maximum(
                    jnp.maximum(feat_scr[pl.ds(mc, MC), :], v) + cb_ref[...],
                    0.0)

    for n in range(4):
        mxu = n % 2
        pltpu.matmul_push_rhs(
            wih1_ref[:, pl.ds(n * 256, 256)].astype(bf16), 0, mxu)
        _mm576(feat_scr, 0, mxu, 0)
        _pop576(xp_scr, n * 256, mxu, b1_ref[0, pl.ds(n * 256, 256)][None, :])

    whh16_scr[...] = jnp.where(d == 0, whh1f_ref[...],
                               whh1b_ref[...]).astype(bf16)

    def store_h(r, t, h):
        l1_ref[0, pl.ds(r, Bp), :] = h

    _lstm_dir_loop(d, xp_scr, whh16_scr, store_h, Bp, H)


def _l2_kernel(l1_ref, wih2f_ref, wih2b_ref, b2_ref, whh2f_ref, whh2b_ref,
               fc1wf_hbm, fc1wb_hbm,
               acc_ref, xp_scr, h2_scr, fc1w_scr, whh16_scr, sem,
               *, Bp, H, FCH):
    f32 = jnp.float32
    d = pl.program_id(0)

    @pl.when(d == 0)
    def _():
        pltpu.make_async_copy(fc1wf_hbm, fc1w_scr, sem).start()

    @pl.when(d != 0)
    def _():
        pltpu.make_async_copy(fc1wb_hbm, fc1w_scr, sem).start()

    for n in range(4):
        mxu = n % 2
        pltpu.matmul_push_rhs(
            wih2f_ref[:, pl.ds(n * 256, 256)].astype(bf16), 0, mxu)
        pltpu.matmul_push_rhs(
            wih2b_ref[:, pl.ds(n * 256, 256)].astype(bf16), 1, mxu)
        _mm576(l1_ref.at[0], 0, mxu, 0)
        _mm576(l1_ref.at[1], 0, mxu, 1)
        _pop576(xp_scr, n * 256, mxu, b2_ref[0, pl.ds(n * 256, 256)][None, :])

    whh16_scr[...] = jnp.where(d == 0, whh2f_ref[...],
                               whh2b_ref[...]).astype(bf16)

    def store_h(r, t, h):
        h2_scr[:, pl.ds(pl.multiple_of(t * H, H), H)] = h

    _lstm_dir_loop(d, xp_scr, whh16_scr, store_h, Bp, H)

    pltpu.make_async_copy(fc1wf_hbm, fc1w_scr, sem).wait()
    for kt in range(T):
        mxu = kt % 2
        msr = (kt // 2) % 2
        pltpu.matmul_push_rhs(
            fc1w_scr[pl.ds(kt * 256, 256), :].astype(bf16), msr, mxu)
        pltpu.matmul_acc_lhs(0, h2_scr[:, pl.ds(kt * 256, 256)].astype(bf16),
                             mxu, load_staged_rhs=msr)
    acc_ref[0] = (pltpu.matmul_pop(0, (Bp, FCH), f32, 0)
                  + pltpu.matmul_pop(0, (Bp, FCH), f32, 1))


def _head_kernel(acc_ref, fc1b_ref, fc2w_ref, fc2b_ref, fc3w_ref, fc3b_ref,
                 o_ref):
    f32 = jnp.float32
    y = jnp.maximum(acc_ref[0] + acc_ref[1] + fc1b_ref[...], 0.0)
    y = jnp.maximum(jnp.dot(y, fc2w_ref[...], preferred_element_type=f32)
                    + fc2b_ref[...], 0.0)
    o_ref[...] = jnp.sum(y * fc3w_ref[...], axis=1, keepdims=True) + fc3b_ref[...]


def kernel(x, cw, cb, wih1, b1, whh1f, whh1b, wih2f, wih2b, b2, whh2f, whh2b,
           fc1wf, fc1wb, fc1b, fc2w, fc2b, fc3w, fc3b):
    f32 = jnp.float32
    B, L, Cin = x.shape
    H = whh1f.shape[0]
    FCH = fc2w.shape[0]
    C = cw.shape[1]
    CK = cw.shape[0]
    Bp = max(8, (B + 7) // 8 * 8)

    xb = jnp.pad(x.astype(f32), ((0, Bp - B), (0, 0), (0, 0)))
    x_bcl = jnp.transpose(xb, (0, 2, 1))

    slist = []
    cwp = [[], [], []]
    r0 = 0
    for K in CONV_KS:
        K2 = K + POOL - 1
        pad_l = (K - 1) // 2
        pad_r = (K - 1) - pad_l
        xpd = jnp.pad(x_bcl, ((0, 0), (0, 0), (pad_l, pad_r)))
        idx = POOL * jnp.arange(T)[:, None] + jnp.arange(K2)[None, :]
        pt = xpd[:, :, idx]
        pt = jnp.transpose(pt, (2, 0, 1, 3)).reshape(T * Bp, Cin * K2)
        slist.append(pt)
        cwb = cw[r0:r0 + Cin * K].reshape(Cin, K, C)
        for p in range(POOL):
            cwp[p].append(
                jnp.pad(cwb, ((0, 0), (p, K2 - K - p), (0, 0))).reshape(
                    Cin * K2, C))
        r0 += Cin * K
    spatch = jnp.concatenate(slist, axis=-1)
    cwp = [jnp.concatenate(c, axis=0) for c in cwp]
    CK2 = spatch.shape[1]

    l1 = pl.pallas_call(
        functools.partial(_l1_kernel, Bp=Bp, H=H),
        out_shape=jax.ShapeDtypeStruct((2, T * Bp, H), f32),
        grid=(2,),
        in_specs=[
            _full((T * Bp, CK2)),
            _full((CK2, C)), _full((CK2, C)), _full((CK2, C)), _full((1, C)),
            pl.BlockSpec((C, 4 * H), lambda d: (0, d)),
            pl.BlockSpec((1, 4 * H), lambda d: (0, d)),
            _full((H, 4 * H)), _full((H, 4 * H)),
        ],
        out_specs=pl.BlockSpec((1, T * Bp, H), lambda d: (d, 0, 0)),
        scratch_shapes=[
            pltpu.VMEM((T * Bp, 256), f32),
            pltpu.VMEM((256, 256), f32),
            pltpu.VMEM((T * Bp, C), f32),
            pltpu.VMEM((T * Bp, 4 * H), f32),
            pltpu.VMEM((H, 4 * H), bf16),
        ],
        compiler_params=pltpu.CompilerParams(
            dimension_semantics=("parallel",)),
    )(spatch, cwp[0], cwp[1], cwp[2], cb, wih1, b1, whh1f, whh1b)

    acc = pl.pallas_call(
        functools.partial(_l2_kernel, Bp=Bp, H=H, FCH=FCH),
        out_shape=jax.ShapeDtypeStruct((2, Bp, FCH), f32),
        grid=(2,),
        in_specs=[
            _full((2, T * Bp, H)),
            pl.BlockSpec((H, 4 * H), lambda d: (0, d)),
            pl.BlockSpec((H, 4 * H), lambda d: (0, d)),
            pl.BlockSpec((1, 4 * H), lambda d: (0, d)),
            _full((H, 4 * H)), _full((H, 4 * H)),
            pl.BlockSpec(memory_space=pl.ANY),
            pl.BlockSpec(memory_space=pl.ANY),
        ],
        out_specs=pl.BlockSpec((1, Bp, FCH), lambda d: (d, 0, 0)),
        scratch_shapes=[
            pltpu.VMEM((T * Bp, 4 * H), f32),
            pltpu.VMEM((Bp, T * H), f32),
            pltpu.VMEM((T * H, FCH), f32),
            pltpu.VMEM((H, 4 * H), bf16),
            pltpu.SemaphoreType.DMA,
        ],
        compiler_params=pltpu.CompilerParams(
            dimension_semantics=("parallel",)),
    )(l1, wih2f, wih2b, b2, whh2f, whh2b, fc1wf, fc1wb)

    out = pl.pallas_call(
        _head_kernel,
        out_shape=jax.ShapeDtypeStruct((Bp, 1), f32),
        grid=(1,),
        in_specs=[
            _full((2, Bp, FCH)), _full((1, FCH)),
            _full((FCH, FCH)), _full((1, FCH)),
            _full((1, FCH)), _full((1, 1)),
        ],
        out_specs=_full((Bp, 1)),
        compiler_params=pltpu.CompilerParams(
            dimension_semantics=("arbitrary",)),
    )(acc, fc1b, fc2w, fc2b, fc3w, fc3b)

    return out[:B, 0]


# speedup vs baseline: 2.0759x; 1.2261x over previous
import functools

import jax
import jax.numpy as jnp
from jax import lax
from jax.experimental import pallas as pl
from jax.experimental.pallas import tpu as pltpu

T = 36
POOL = 3
CONV_KS = (10, 15)
MC = 144
bf16 = jnp.bfloat16


def _sigmoid(x):
    return 0.5 * (jnp.tanh(0.5 * x) + 1.0)


def _full(shape):
    nd = len(shape)
    return pl.BlockSpec(tuple(shape), lambda _n=nd: (0,) * _n)


def _mm576(lhs_ref, col0, mxu, lsr):
    for j, mc in enumerate(range(0, T * 16, MC)):
        chunk = lhs_ref[pl.ds(mc, MC), pl.ds(col0, 256)].astype(bf16)
        pltpu.matmul_acc_lhs(mc // 4, chunk, mxu,
                             load_staged_rhs=lsr if j == 0 else None)


def _pop576(out_ref, col0, mxu, bias):
    for mc in range(0, T * 16, MC):
        v = pltpu.matmul_pop(mc // 4, (MC, 256), jnp.float32, mxu)
        out_ref[pl.ds(mc, MC), pl.ds(col0, 256)] = v + bias


def _cell(g0, g1, g2, g3, c_prev):
    i = _sigmoid(g0)
    f = _sigmoid(g1)
    g = jnp.tanh(g2)
    o = _sigmoid(g3)
    c = f * c_prev + i * g
    return o * jnp.tanh(c), c


def _lstm_bidir_loop(xp_scr, whh16_scr, store_fwd, store_bwd, Bp, H):
    f32 = jnp.float32
    z = jnp.zeros((Bp, H), f32)

    def body(s, carry):
        hf, cf, hb, cb = carry
        rf = pl.multiple_of(s * Bp, Bp)
        rb = pl.multiple_of((T - 1 - s) * Bp, Bp)
        hf16 = hf.astype(bf16)
        hb16 = hb.astype(bf16)
        for mxu in range(2):
            t0 = 2 * mxu
            pltpu.matmul_push_rhs(whh16_scr[0, :, pl.ds(t0 * 256, 256)], 0, mxu)
            pltpu.matmul_acc_lhs(0, hf16, mxu, load_staged_rhs=0)
            pltpu.matmul_push_rhs(whh16_scr[0, :, pl.ds((t0 + 1) * 256, 256)], 1, mxu)
            pltpu.matmul_acc_lhs(8, hf16, mxu, load_staged_rhs=1)
            pltpu.matmul_push_rhs(whh16_scr[1, :, pl.ds(t0 * 256, 256)], 0, mxu)
            pltpu.matmul_acc_lhs(16, hb16, mxu, load_staged_rhs=0)
            pltpu.matmul_push_rhs(whh16_scr[1, :, pl.ds((t0 + 1) * 256, 256)], 1, mxu)
            pltpu.matmul_acc_lhs(24, hb16, mxu, load_staged_rhs=1)
        xpf = xp_scr[pl.ds(rf, Bp), :]
        xpb = xp_scr[pl.ds(rb, Bp), :]
        gf0 = pltpu.matmul_pop(0, (Bp, 256), f32, 0) + xpf[:, 0:256]
        gf1 = pltpu.matmul_pop(8, (Bp, 256), f32, 0) + xpf[:, 256:512]
        gf2 = pltpu.matmul_pop(0, (Bp, 256), f32, 1) + xpf[:, 512:768]
        gf3 = pltpu.matmul_pop(8, (Bp, 256), f32, 1) + xpf[:, 768:1024]
        hf, cf = _cell(gf0, gf1, gf2, gf3, cf)
        store_fwd(rf, s, hf)
        gb0 = pltpu.matmul_pop(16, (Bp, 256), f32, 0) + xpb[:, 1024:1280]
        gb1 = pltpu.matmul_pop(24, (Bp, 256), f32, 0) + xpb[:, 1280:1536]
        gb2 = pltpu.matmul_pop(16, (Bp, 256), f32, 1) + xpb[:, 1536:1792]
        gb3 = pltpu.matmul_pop(24, (Bp, 256), f32, 1) + xpb[:, 1792:2048]
        hb, cb = _cell(gb0, gb1, gb2, gb3, cb)
        store_bwd(rb, T - 1 - s, hb)
        return hf, cf, hb, cb

    lax.fori_loop(0, T, body, (z, z, z, z))


def _l1_kernel(sp_ref, cwp0_ref, cwp1_ref, cwp2_ref, cb_ref,
               wih1_ref, b1_ref, whh1f_ref, whh1b_ref,
               l1_ref, pscr, cwscr, feat_scr, xp_scr, whh16_scr, *, Bp, H):
    f32 = jnp.float32
    CK2 = cwp0_ref.shape[0]

    pscr[...] = jnp.zeros((T * Bp, 256), f32)
    pscr[:, pl.ds(0, CK2)] = sp_ref[...]
    cwscr[...] = jnp.zeros((256, 256), f32)

    for p, w_ref in enumerate((cwp0_ref, cwp1_ref, cwp2_ref)):
        cwscr[pl.ds(0, CK2), :] = w_ref[...]
        mxu = p % 2
        pltpu.matmul_push_rhs(cwscr[...].astype(bf16), 0, mxu)
        _mm576(pscr, 0, mxu, 0)
        for mc in range(0, T * Bp, MC):
            v = pltpu.matmul_pop(mc // 4, (MC, 256), f32, mxu)
            if p == 0:
                feat_scr[pl.ds(mc, MC), :] = v
            elif p == 1:
                feat_scr[pl.ds(mc, MC), :] = jnp.maximum(
                    feat_scr[pl.ds(mc, MC), :], v)
            else:
                feat_scr[pl.ds(mc, MC), :] = jnp.maximum(
                    jnp.maximum(feat_scr[pl.ds(mc, MC), :], v) + cb_ref[...],
                    0.0)

    for n in range(8):
        mxu = n % 2
        pltpu.matmul_push_rhs(
            wih1_ref[:, pl.ds(n * 256, 256)].astype(bf16), 0, mxu)
        _mm576(feat_scr, 0, mxu, 0)
        _pop576(xp_scr, n * 256, mxu, b1_ref[0, pl.ds(n * 256, 256)][None, :])

    whh16_scr[0] = whh1f_ref[...].astype(bf16)
    whh16_scr[1] = whh1b_ref[...].astype(bf16)

    def store_fwd(r, t, h):
        l1_ref[0, pl.ds(r, Bp), :] = h

    def store_bwd(r, t, h):
        l1_ref[1, pl.ds(r, Bp), :] = h

    _lstm_bidir_loop(xp_scr, whh16_scr, store_fwd, store_bwd, Bp, H)


def _l2_kernel(l1_ref, wih2f_ref, wih2b_ref, b2_ref, whh2f_ref, whh2b_ref,
               fc1wf_hbm, fc1wb_hbm,
               acc_ref, xp_scr, h2f_scr, h2b_scr, fc1wf_scr, fc1wb_scr,
               whh16_scr, semf, semb, *, Bp, H, FCH):
    f32 = jnp.float32

    pltpu.make_async_copy(fc1wf_hbm, fc1wf_scr, semf).start()
    pltpu.make_async_copy(fc1wb_hbm, fc1wb_scr, semb).start()

    for n in range(8):
        mxu = n % 2
        pltpu.matmul_push_rhs(
            wih2f_ref[:, pl.ds(n * 256, 256)].astype(bf16), 0, mxu)
        pltpu.matmul_push_rhs(
            wih2b_ref[:, pl.ds(n * 256, 256)].astype(bf16), 1, mxu)
        _mm576(l1_ref.at[0], 0, mxu, 0)
        _mm576(l1_ref.at[1], 0, mxu, 1)
        _pop576(xp_scr, n * 256, mxu, b2_ref[0, pl.ds(n * 256, 256)][None, :])

    whh16_scr[0] = whh2f_ref[...].astype(bf16)
    whh16_scr[1] = whh2b_ref[...].astype(bf16)

    def store_fwd(r, t, h):
        h2f_scr[:, pl.ds(pl.multiple_of(t * H, H), H)] = h

    def store_bwd(r, t, h):
        h2b_scr[:, pl.ds(pl.multiple_of(t * H, H), H)] = h

    _lstm_bidir_loop(xp_scr, whh16_scr, store_fwd, store_bwd, Bp, H)

    pltpu.make_async_copy(fc1wf_hbm, fc1wf_scr, semf).wait()
    pltpu.make_async_copy(fc1wb_hbm, fc1wb_scr, semb).wait()
    for kt in range(T):
        msr = kt % 2
        pltpu.matmul_push_rhs(
            fc1wf_scr[pl.ds(kt * 256, 256), :].astype(bf16), msr, 0)
        pltpu.matmul_acc_lhs(0, h2f_scr[:, pl.ds(kt * 256, 256)].astype(bf16),
                             0, load_staged_rhs=msr)
        pltpu.matmul_push_rhs(
            fc1wb_scr[pl.ds(kt * 256, 256), :].astype(bf16), msr, 1)
        pltpu.matmul_acc_lhs(0, h2b_scr[:, pl.ds(kt * 256, 256)].astype(bf16),
                             1, load_staged_rhs=msr)
    acc_ref[...] = (pltpu.matmul_pop(0, (Bp, FCH), f32, 0)
                    + pltpu.matmul_pop(0, (Bp, FCH), f32, 1))


def _head_kernel(acc_ref, fc1b_ref, fc2w_ref, fc2b_ref, fc3w_ref, fc3b_ref,
                 o_ref):
    f32 = jnp.float32
    y = jnp.maximum(acc_ref[...] + fc1b_ref[...], 0.0)
    y = jnp.maximum(jnp.dot(y, fc2w_ref[...], preferred_element_type=f32)
                    + fc2b_ref[...], 0.0)
    o_ref[...] = jnp.sum(y * fc3w_ref[...], axis=1, keepdims=True) + fc3b_ref[...]


def kernel(x, cw, cb, wih1, b1, whh1f, whh1b, wih2f, wih2b, b2, whh2f, whh2b,
           fc1wf, fc1wb, fc1b, fc2w, fc2b, fc3w, fc3b):
    f32 = jnp.float32
    B, L, Cin = x.shape
    H = whh1f.shape[0]
    FCH = fc2w.shape[0]
    C = cw.shape[1]
    Bp = max(8, (B + 7) // 8 * 8)

    xb = jnp.pad(x.astype(f32), ((0, Bp - B), (0, 0), (0, 0)))
    x_bcl = jnp.transpose(xb, (0, 2, 1))

    slist = []
    cwp = [[], [], []]
    r0 = 0
    for K in CONV_KS:
        K2 = K + POOL - 1
        pad_l = (K - 1) // 2
        pad_r = (K - 1) - pad_l
        xpd = jnp.pad(x_bcl, ((0, 0), (0, 0), (pad_l, pad_r)))
        idx = POOL * jnp.arange(T)[:, None] + jnp.arange(K2)[None, :]
        pt = xpd[:, :, idx]
        pt = jnp.transpose(pt, (2, 0, 1, 3)).reshape(T * Bp, Cin * K2)
        slist.append(pt)
        cwb = cw[r0:r0 + Cin * K].reshape(Cin, K, C)
        for p in range(POOL):
            cwp[p].append(
                jnp.pad(cwb, ((0, 0), (p, K2 - K - p), (0, 0))).reshape(
                    Cin * K2, C))
        r0 += Cin * K
    spatch = jnp.concatenate(slist, axis=-1)
    cwp = [jnp.concatenate(c, axis=0) for c in cwp]
    CK2 = spatch.shape[1]

    l1 = pl.pallas_call(
        functools.partial(_l1_kernel, Bp=Bp, H=H),
        out_shape=jax.ShapeDtypeStruct((2, T * Bp, H), f32),
        in_specs=[
            _full((T * Bp, CK2)),
            _full((CK2, C)), _full((CK2, C)), _full((CK2, C)), _full((1, C)),
            _full((C, 8 * H)), _full((1, 8 * H)),
            _full((H, 4 * H)), _full((H, 4 * H)),
        ],
        out_specs=_full((2, T * Bp, H)),
        scratch_shapes=[
            pltpu.VMEM((T * Bp, 256), f32),
            pltpu.VMEM((256, 256), f32),
            pltpu.VMEM((T * Bp, C), f32),
            pltpu.VMEM((T * Bp, 8 * H), f32),
            pltpu.VMEM((2, H, 4 * H), bf16),
        ],
        grid=(),
    )(spatch, cwp[0], cwp[1], cwp[2], cb, wih1, b1, whh1f, whh1b)

    acc = pl.pallas_call(
        functools.partial(_l2_kernel, Bp=Bp, H=H, FCH=FCH),
        out_shape=jax.ShapeDtypeStruct((Bp, FCH), f32),
        in_specs=[
            _full((2, T * Bp, H)),
            _full((H, 8 * H)), _full((H, 8 * H)), _full((1, 8 * H)),
            _full((H, 4 * H)), _full((H, 4 * H)),
            pl.BlockSpec(memory_space=pl.ANY),
            pl.BlockSpec(memory_space=pl.ANY),
        ],
        out_specs=_full((Bp, FCH)),
        scratch_shapes=[
            pltpu.VMEM((T * Bp, 8 * H), f32),
            pltpu.VMEM((Bp, T * H), f32),
            pltpu.VMEM((Bp, T * H), f32),
            pltpu.VMEM((T * H, FCH), f32),
            pltpu.VMEM((T * H, FCH), f32),
            pltpu.VMEM((2, H, 4 * H), bf16),
            pltpu.SemaphoreType.DMA,
            pltpu.SemaphoreType.DMA,
        ],
        grid=(),
    )(l1, wih2f, wih2b, b2, whh2f, whh2b, fc1wf, fc1wb)

    out = pl.pallas_call(
        _head_kernel,
        out_shape=jax.ShapeDtypeStruct((Bp, 1), f32),
        in_specs=[
            _full((Bp, FCH)), _full((1, FCH)),
            _full((FCH, FCH)), _full((1, FCH)),
            _full((1, FCH)), _full((1, 1)),
        ],
        out_specs=_full((Bp, 1)),
        grid=(),
    )(acc, fc1b, fc2w, fc2b, fc3w, fc3b)

    return out[:B, 0]


# speedup vs baseline: 2.2138x; 1.0664x over previous
import functools

import jax
import jax.numpy as jnp
from jax import lax
from jax.experimental import pallas as pl
from jax.experimental.pallas import tpu as pltpu

T = 36
POOL = 3
CONV_KS = (10, 15)
MC = 144
bf16 = jnp.bfloat16


def _sigmoid(x):
    return 0.5 * (jnp.tanh(0.5 * x) + 1.0)


def _full(shape):
    nd = len(shape)
    return pl.BlockSpec(tuple(shape), lambda _n=nd: (0,) * _n)


def _mm576(lhs_ref, col0, mxu, lsr):
    for j, mc in enumerate(range(0, T * 16, MC)):
        chunk = lhs_ref[pl.ds(mc, MC), pl.ds(col0, 256)].astype(bf16)
        pltpu.matmul_acc_lhs(mc // 4, chunk, mxu,
                             load_staged_rhs=lsr if j == 0 else None)


def _pop576(out_ref, col0, mxu, bias):
    for mc in range(0, T * 16, MC):
        v = pltpu.matmul_pop(mc // 4, (MC, 256), jnp.float32, mxu)
        out_ref[pl.ds(mc, MC), pl.ds(col0, 256)] = v + bias


def _cell(g0, g1, g2, g3, c_prev):
    i = _sigmoid(g0)
    f = _sigmoid(g1)
    g = jnp.tanh(g2)
    o = _sigmoid(g3)
    c = f * c_prev + i * g
    return o * jnp.tanh(c), c


def _lstm_bidir_loop(xp_scr, whh16_scr, store_fwd, store_bwd, Bp, H):
    f32 = jnp.float32
    z = jnp.zeros((Bp, H), f32)

    def body(s, carry):
        hf, cf, hb, cb = carry
        rf = pl.multiple_of(s * Bp, Bp)
        rb = pl.multiple_of((T - 1 - s) * Bp, Bp)
        hf16 = hf.astype(bf16)
        hb16 = hb.astype(bf16)
        for mxu in range(2):
            t0 = 2 * mxu
            pltpu.matmul_push_rhs(whh16_scr[0, :, pl.ds(t0 * 256, 256)], 0, mxu)
            pltpu.matmul_acc_lhs(0, hf16, mxu, load_staged_rhs=0)
            pltpu.matmul_push_rhs(whh16_scr[0, :, pl.ds((t0 + 1) * 256, 256)], 1, mxu)
            pltpu.matmul_acc_lhs(8, hf16, mxu, load_staged_rhs=1)
            pltpu.matmul_push_rhs(whh16_scr[1, :, pl.ds(t0 * 256, 256)], 0, mxu)
            pltpu.matmul_acc_lhs(16, hb16, mxu, load_staged_rhs=0)
            pltpu.matmul_push_rhs(whh16_scr[1, :, pl.ds((t0 + 1) * 256, 256)], 1, mxu)
            pltpu.matmul_acc_lhs(24, hb16, mxu, load_staged_rhs=1)
        xpf = xp_scr[pl.ds(rf, Bp), :]
        xpb = xp_scr[pl.ds(rb, Bp), :]
        gf0 = pltpu.matmul_pop(0, (Bp, 256), f32, 0) + xpf[:, 0:256]
        gf1 = pltpu.matmul_pop(8, (Bp, 256), f32, 0) + xpf[:, 256:512]
        gf2 = pltpu.matmul_pop(0, (Bp, 256), f32, 1) + xpf[:, 512:768]
        gf3 = pltpu.matmul_pop(8, (Bp, 256), f32, 1) + xpf[:, 768:1024]
        hf, cf = _cell(gf0, gf1, gf2, gf3, cf)
        store_fwd(rf, s, hf)
        gb0 = pltpu.matmul_pop(16, (Bp, 256), f32, 0) + xpb[:, 1024:1280]
        gb1 = pltpu.matmul_pop(24, (Bp, 256), f32, 0) + xpb[:, 1280:1536]
        gb2 = pltpu.matmul_pop(16, (Bp, 256), f32, 1) + xpb[:, 1536:1792]
        gb3 = pltpu.matmul_pop(24, (Bp, 256), f32, 1) + xpb[:, 1792:2048]
        hb, cb = _cell(gb0, gb1, gb2, gb3, cb)
        store_bwd(rb, T - 1 - s, hb)
        return hf, cf, hb, cb

    lax.fori_loop(0, T, body, (z, z, z, z))


def _l1_kernel(sp_ref, cwp0_ref, cwp1_ref, cwp2_ref, cb_ref,
               wih1_ref, b1_ref, whh1f_ref, whh1b_ref,
               l1_ref, pscr, cwscr, feat_scr, xp_scr, whh16_scr, *, Bp, H):
    f32 = jnp.float32
    CK2 = cwp0_ref.shape[0]

    pscr[...] = jnp.zeros((T * Bp, 256), f32)
    pscr[:, pl.ds(0, CK2)] = sp_ref[...]
    cwscr[...] = jnp.zeros((256, 256), f32)

    for p, w_ref in enumerate((cwp0_ref, cwp1_ref, cwp2_ref)):
        cwscr[pl.ds(0, CK2), :] = w_ref[...]
        mxu = p % 2
        pltpu.matmul_push_rhs(cwscr[...].astype(bf16), 0, mxu)
        _mm576(pscr, 0, mxu, 0)
        for mc in range(0, T * Bp, MC):
            v = pltpu.matmul_pop(mc // 4, (MC, 256), f32, mxu)
            if p == 0:
                feat_scr[pl.ds(mc, MC), :] = v
            elif p == 1:
                feat_scr[pl.ds(mc, MC), :] = jnp.maximum(
                    feat_scr[pl.ds(mc, MC), :], v)
            else:
                feat_scr[pl.ds(mc, MC), :] = jnp.maximum(
                    jnp.maximum(feat_scr[pl.ds(mc, MC), :], v) + cb_ref[...],
                    0.0)

    for n in range(8):
        mxu = n % 2
        pltpu.matmul_push_rhs(
            wih1_ref[:, pl.ds(n * 256, 256)].astype(bf16), 0, mxu)
        _mm576(feat_scr, 0, mxu, 0)
        _pop576(xp_scr, n * 256, mxu, b1_ref[0, pl.ds(n * 256, 256)][None, :])

    whh16_scr[0] = whh1f_ref[...].astype(bf16)
    whh16_scr[1] = whh1b_ref[...].astype(bf16)

    def store_fwd(r, t, h):
        l1_ref[0, pl.ds(r, Bp), :] = h

    def store_bwd(r, t, h):
        l1_ref[1, pl.ds(r, Bp), :] = h

    _lstm_bidir_loop(xp_scr, whh16_scr, store_fwd, store_bwd, Bp, H)


def _l2_kernel(l1_ref, wih2f_ref, wih2b_ref, b2_ref, whh2f_ref, whh2b_ref,
               fc1wf_hbm, fc1wb_hbm,
               acc_ref, xp_scr, h2f_scr, h2b_scr, fc1wf_scr, fc1wb_scr,
               whh16_scr, semf, semb, *, Bp, H, FCH):
    f32 = jnp.float32

    pltpu.make_async_copy(fc1wf_hbm, fc1wf_scr, semf).start()
    pltpu.make_async_copy(fc1wb_hbm, fc1wb_scr, semb).start()

    for n in range(8):
        mxu = n % 2
        pltpu.matmul_push_rhs(
            wih2f_ref[:, pl.ds(n * 256, 256)].astype(bf16), 0, mxu)
        pltpu.matmul_push_rhs(
            wih2b_ref[:, pl.ds(n * 256, 256)].astype(bf16), 1, mxu)
        _mm576(l1_ref.at[0], 0, mxu, 0)
        _mm576(l1_ref.at[1], 0, mxu, 1)
        _pop576(xp_scr, n * 256, mxu, b2_ref[0, pl.ds(n * 256, 256)][None, :])

    whh16_scr[0] = whh2f_ref[...].astype(bf16)
    whh16_scr[1] = whh2b_ref[...].astype(bf16)

    def store_fwd(r, t, h):
        h2f_scr[:, pl.ds(pl.multiple_of(t * H, H), H)] = h

    def store_bwd(r, t, h):
        h2b_scr[:, pl.ds(pl.multiple_of(t * H, H), H)] = h

    _lstm_bidir_loop(xp_scr, whh16_scr, store_fwd, store_bwd, Bp, H)

    pltpu.make_async_copy(fc1wf_hbm, fc1wf_scr, semf).wait()
    pltpu.make_async_copy(fc1wb_hbm, fc1wb_scr, semb).wait()
    for kt in range(T):
        msr = kt % 2
        pltpu.matmul_push_rhs(
            fc1wf_scr[pl.ds(kt * 256, 256), :].astype(bf16), msr, 0)
        pltpu.matmul_acc_lhs(0, h2f_scr[:, pl.ds(kt * 256, 256)].astype(bf16),
                             0, load_staged_rhs=msr)
        pltpu.matmul_push_rhs(
            fc1wb_scr[pl.ds(kt * 256, 256), :].astype(bf16), msr, 1)
        pltpu.matmul_acc_lhs(0, h2b_scr[:, pl.ds(kt * 256, 256)].astype(bf16),
                             1, load_staged_rhs=msr)
    acc_ref[...] = (pltpu.matmul_pop(0, (Bp, FCH), f32, 0)
                    + pltpu.matmul_pop(0, (Bp, FCH), f32, 1))


def _head_kernel(acc_ref, fc1b_ref, fc2w_ref, fc2b_ref, fc3w_ref, fc3b_ref,
                 o_ref):
    f32 = jnp.float32
    y = jnp.maximum(acc_ref[...] + fc1b_ref[...], 0.0)
    y = jnp.maximum(jnp.dot(y, fc2w_ref[...], preferred_element_type=f32)
                    + fc2b_ref[...], 0.0)
    o_ref[...] = jnp.sum(y * fc3w_ref[...], axis=1, keepdims=True) + fc3b_ref[...]


def kernel(x, cw, cb, wih1, b1, whh1f, whh1b, wih2f, wih2b, b2, whh2f, whh2b,
           fc1wf, fc1wb, fc1b, fc2w, fc2b, fc3w, fc3b):
    f32 = jnp.float32
    B, L, Cin = x.shape
    H = whh1f.shape[0]
    FCH = fc2w.shape[0]
    C = cw.shape[1]
    Bp = max(8, (B + 7) // 8 * 8)

    xb = jnp.pad(x.astype(f32), ((0, Bp - B), (0, 0), (0, 0)))
    x_bcl = jnp.transpose(xb, (0, 2, 1))

    Kmax = max(CONV_KS)
    K2 = Kmax + POOL - 1
    pad_big = (Kmax - 1) // 2
    xpd = jnp.pad(x_bcl, ((0, 0), (0, 0), (pad_big, Kmax - 1 - pad_big)))
    idx = POOL * jnp.arange(T)[:, None] + jnp.arange(K2)[None, :]
    pt = xpd[:, :, idx]
    spatch = jnp.transpose(pt, (2, 0, 1, 3)).reshape(T * Bp, Cin * K2)
    cwp = []
    for p in range(POOL):
        w_p = jnp.zeros((Cin, K2, C), f32)
        r0 = 0
        for K in CONV_KS:
            off = pad_big - (K - 1) // 2
            cwb = cw[r0:r0 + Cin * K].reshape(Cin, K, C)
            w_p = w_p + jnp.pad(
                cwb, ((0, 0), (p + off, K2 - K - p - off), (0, 0)))
            r0 += Cin * K
        cwp.append(w_p.reshape(Cin * K2, C))
    CK2 = Cin * K2

    l1 = pl.pallas_call(
        functools.partial(_l1_kernel, Bp=Bp, H=H),
        out_shape=jax.ShapeDtypeStruct((2, T * Bp, H), f32),
        in_specs=[
            _full((T * Bp, CK2)),
            _full((CK2, C)), _full((CK2, C)), _full((CK2, C)), _full((1, C)),
            _full((C, 8 * H)), _full((1, 8 * H)),
            _full((H, 4 * H)), _full((H, 4 * H)),
        ],
        out_specs=_full((2, T * Bp, H)),
        scratch_shapes=[
            pltpu.VMEM((T * Bp, 256), f32),
            pltpu.VMEM((256, 256), f32),
            pltpu.VMEM((T * Bp, C), f32),
            pltpu.VMEM((T * Bp, 8 * H), f32),
            pltpu.VMEM((2, H, 4 * H), bf16),
        ],
        grid=(),
    )(spatch, cwp[0], cwp[1], cwp[2], cb, wih1, b1, whh1f, whh1b)

    acc = pl.pallas_call(
        functools.partial(_l2_kernel, Bp=Bp, H=H, FCH=FCH),
        out_shape=jax.ShapeDtypeStruct((Bp, FCH), f32),
        in_specs=[
            _full((2, T * Bp, H)),
            _full((H, 8 * H)), _full((H, 8 * H)), _full((1, 8 * H)),
            _full((H, 4 * H)), _full((H, 4 * H)),
            pl.BlockSpec(memory_space=pl.ANY),
            pl.BlockSpec(memory_space=pl.ANY),
        ],
        out_specs=_full((Bp, FCH)),
        scratch_shapes=[
            pltpu.VMEM((T * Bp, 8 * H), f32),
            pltpu.VMEM((Bp, T * H), f32),
            pltpu.VMEM((Bp, T * H), f32),
            pltpu.VMEM((T * H, FCH), f32),
            pltpu.VMEM((T * H, FCH), f32),
            pltpu.VMEM((2, H, 4 * H), bf16),
            pltpu.SemaphoreType.DMA,
            pltpu.SemaphoreType.DMA,
        ],
        grid=(),
    )(l1, wih2f, wih2b, b2, whh2f, whh2b, fc1wf, fc1wb)

    out = pl.pallas_call(
        _head_kernel,
        out_shape=jax.ShapeDtypeStruct((Bp, 1), f32),
        in_specs=[
            _full((Bp, FCH)), _full((1, FCH)),
            _full((FCH, FCH)), _full((1, FCH)),
            _full((1, FCH)), _full((1, 1)),
        ],
        out_specs=_full((Bp, 1)),
        grid=(),
    )(acc, fc1b, fc2w, fc2b, fc3w, fc3b)

    return out[:B, 0]


# speedup vs baseline: 2.4259x; 1.0958x over previous
import functools

import jax
import jax.numpy as jnp
from jax import lax
from jax.experimental import pallas as pl
from jax.experimental.pallas import tpu as pltpu

T = 36
POOL = 3
CONV_KS = (10, 15)
MC = 144
bf16 = jnp.bfloat16


def _sigmoid(x):
    return 0.5 * (jnp.tanh(0.5 * x) + 1.0)


def _full(shape):
    nd = len(shape)
    return pl.BlockSpec(tuple(shape), lambda _n=nd: (0,) * _n)


def _mm576(lhs_ref, col0, mxu, lsr):
    for j, mc in enumerate(range(0, T * 16, MC)):
        chunk = lhs_ref[pl.ds(mc, MC), pl.ds(col0, 256)].astype(bf16)
        pltpu.matmul_acc_lhs(mc // 4, chunk, mxu,
                             load_staged_rhs=lsr if j == 0 else None)


def _pop576(out_ref, col0, mxu, bias):
    for mc in range(0, T * 16, MC):
        v = pltpu.matmul_pop(mc // 4, (MC, 256), jnp.float32, mxu)
        out_ref[pl.ds(mc, MC), pl.ds(col0, 256)] = v + bias


def _cell(g0, g1, g2, g3, c_prev):
    i = _sigmoid(g0)
    f = _sigmoid(g1)
    g = jnp.tanh(g2)
    o = _sigmoid(g3)
    c = f * c_prev + i * g
    return o * jnp.tanh(c), c


def _lstm_bidir_loop(xp_scr, whh16_scr, store_fwd, store_bwd, Bp, H):
    f32 = jnp.float32
    z = jnp.zeros((Bp, H), f32)

    def body(s, carry):
        hf, cf, hb, cb = carry
        rf = pl.multiple_of(s * Bp, Bp)
        rb = pl.multiple_of((T - 1 - s) * Bp, Bp)
        hf16 = hf.astype(bf16)
        hb16 = hb.astype(bf16)
        for mxu in range(2):
            t0 = 2 * mxu
            pltpu.matmul_push_rhs(whh16_scr[0, :, pl.ds(t0 * 256, 256)], 0, mxu)
            pltpu.matmul_acc_lhs(0, hf16, mxu, load_staged_rhs=0)
            pltpu.matmul_push_rhs(whh16_scr[0, :, pl.ds((t0 + 1) * 256, 256)], 1, mxu)
            pltpu.matmul_acc_lhs(8, hf16, mxu, load_staged_rhs=1)
            pltpu.matmul_push_rhs(whh16_scr[1, :, pl.ds(t0 * 256, 256)], 0, mxu)
            pltpu.matmul_acc_lhs(16, hb16, mxu, load_staged_rhs=0)
            pltpu.matmul_push_rhs(whh16_scr[1, :, pl.ds((t0 + 1) * 256, 256)], 1, mxu)
            pltpu.matmul_acc_lhs(24, hb16, mxu, load_staged_rhs=1)
        xpf = xp_scr[pl.ds(rf, Bp), :]
        xpb = xp_scr[pl.ds(rb, Bp), :]
        gf0 = pltpu.matmul_pop(0, (Bp, 256), f32, 0) + xpf[:, 0:256]
        gf1 = pltpu.matmul_pop(8, (Bp, 256), f32, 0) + xpf[:, 256:512]
        gf2 = pltpu.matmul_pop(0, (Bp, 256), f32, 1) + xpf[:, 512:768]
        gf3 = pltpu.matmul_pop(8, (Bp, 256), f32, 1) + xpf[:, 768:1024]
        hf, cf = _cell(gf0, gf1, gf2, gf3, cf)
        store_fwd(rf, s, hf)
        gb0 = pltpu.matmul_pop(16, (Bp, 256), f32, 0) + xpb[:, 1024:1280]
        gb1 = pltpu.matmul_pop(24, (Bp, 256), f32, 0) + xpb[:, 1280:1536]
        gb2 = pltpu.matmul_pop(16, (Bp, 256), f32, 1) + xpb[:, 1536:1792]
        gb3 = pltpu.matmul_pop(24, (Bp, 256), f32, 1) + xpb[:, 1792:2048]
        hb, cb = _cell(gb0, gb1, gb2, gb3, cb)
        store_bwd(rb, T - 1 - s, hb)
        return hf, cf, hb, cb

    lax.fori_loop(0, T, body, (z, z, z, z))


def _fused_kernel(sp_ref, cwp0_ref, cwp1_ref, cwp2_ref, cb_ref,
                  wih1_ref, b1_ref, whh1f_ref, whh1b_ref,
                  wih2f_ref, wih2b_ref, b2_ref, whh2f_ref, whh2b_ref,
                  fc1wf_hbm, fc1wb_hbm, fc1b_ref,
                  fc2w_ref, fc2b_ref, fc3w_ref, fc3b_ref,
                  o_ref,
                  pscr, cwscr, feat_scr, xp_scr, whh16_scr,
                  l1_scr, h2f_scr, h2b_scr, fc1wf_scr, fc1wb_scr,
                  semf, semb, *, Bp, H, FCH):
    f32 = jnp.float32
    CK2 = cwp0_ref.shape[0]

    pltpu.make_async_copy(fc1wf_hbm, fc1wf_scr, semf).start()
    pltpu.make_async_copy(fc1wb_hbm, fc1wb_scr, semb).start()

    pscr[...] = jnp.zeros((T * Bp, 256), f32)
    pscr[:, pl.ds(0, CK2)] = sp_ref[...]
    cwscr[...] = jnp.zeros((256, 256), f32)

    for p, w_ref in enumerate((cwp0_ref, cwp1_ref, cwp2_ref)):
        cwscr[pl.ds(0, CK2), :] = w_ref[...]
        mxu = p % 2
        pltpu.matmul_push_rhs(cwscr[...].astype(bf16), 0, mxu)
        _mm576(pscr, 0, mxu, 0)
        for mc in range(0, T * Bp, MC):
            v = pltpu.matmul_pop(mc // 4, (MC, 256), f32, mxu)
            if p == 0:
                feat_scr[pl.ds(mc, MC), :] = v
            elif p == 1:
                feat_scr[pl.ds(mc, MC), :] = jnp.maximum(
                    feat_scr[pl.ds(mc, MC), :], v)
            else:
                feat_scr[pl.ds(mc, MC), :] = jnp.maximum(
                    jnp.maximum(feat_scr[pl.ds(mc, MC), :], v) + cb_ref[...],
                    0.0)

    for n in range(8):
        mxu = n % 2
        pltpu.matmul_push_rhs(
            wih1_ref[:, pl.ds(n * 256, 256)].astype(bf16), 0, mxu)
        _mm576(feat_scr, 0, mxu, 0)
        _pop576(xp_scr, n * 256, mxu, b1_ref[0, pl.ds(n * 256, 256)][None, :])

    whh16_scr[0] = whh1f_ref[...].astype(bf16)
    whh16_scr[1] = whh1b_ref[...].astype(bf16)

    def store_fwd1(r, t, h):
        l1_scr[0, pl.ds(r, Bp), :] = h

    def store_bwd1(r, t, h):
        l1_scr[1, pl.ds(r, Bp), :] = h

    _lstm_bidir_loop(xp_scr, whh16_scr, store_fwd1, store_bwd1, Bp, H)

    for n in range(8):
        mxu = n % 2
        pltpu.matmul_push_rhs(
            wih2f_ref[:, pl.ds(n * 256, 256)].astype(bf16), 0, mxu)
        pltpu.matmul_push_rhs(
            wih2b_ref[:, pl.ds(n * 256, 256)].astype(bf16), 1, mxu)
        _mm576(l1_scr.at[0], 0, mxu, 0)
        _mm576(l1_scr.at[1], 0, mxu, 1)
        _pop576(xp_scr, n * 256, mxu, b2_ref[0, pl.ds(n * 256, 256)][None, :])

    whh16_scr[0] = whh2f_ref[...].astype(bf16)
    whh16_scr[1] = whh2b_ref[...].astype(bf16)

    def store_fwd(r, t, h):
        h2f_scr[:, pl.ds(pl.multiple_of(t * H, H), H)] = h

    def store_bwd(r, t, h):
        h2b_scr[:, pl.ds(pl.multiple_of(t * H, H), H)] = h

    _lstm_bidir_loop(xp_scr, whh16_scr, store_fwd, store_bwd, Bp, H)

    pltpu.make_async_copy(fc1wf_hbm, fc1wf_scr, semf).wait()
    pltpu.make_async_copy(fc1wb_hbm, fc1wb_scr, semb).wait()
    for kt in range(T):
        msr = kt % 2
        pltpu.matmul_push_rhs(
            fc1wf_scr[pl.ds(kt * 256, 256), :].astype(bf16), msr, 0)
        pltpu.matmul_acc_lhs(0, h2f_scr[:, pl.ds(kt * 256, 256)].astype(bf16),
                             0, load_staged_rhs=msr)
        pltpu.matmul_push_rhs(
            fc1wb_scr[pl.ds(kt * 256, 256), :].astype(bf16), msr, 1)
        pltpu.matmul_acc_lhs(0, h2b_scr[:, pl.ds(kt * 256, 256)].astype(bf16),
                             1, load_staged_rhs=msr)
    acc = (pltpu.matmul_pop(0, (Bp, FCH), f32, 0)
           + pltpu.matmul_pop(0, (Bp, FCH), f32, 1))

    y = jnp.maximum(acc + fc1b_ref[...], 0.0)
    pltpu.matmul_push_rhs(fc2w_ref[...].astype(bf16), 0, 0)
    pltpu.matmul_acc_lhs(0, y.astype(bf16), 0, load_staged_rhs=0)
    y = jnp.maximum(pltpu.matmul_pop(0, (Bp, FCH), f32, 0)
                    + fc2b_ref[...], 0.0)
    o_ref[...] = jnp.sum(y * fc3w_ref[...], axis=1, keepdims=True) + fc3b_ref[...]


def kernel(x, cw, cb, wih1, b1, whh1f, whh1b, wih2f, wih2b, b2, whh2f, whh2b,
           fc1wf, fc1wb, fc1b, fc2w, fc2b, fc3w, fc3b):
    f32 = jnp.float32
    B, L, Cin = x.shape
    H = whh1f.shape[0]
    FCH = fc2w.shape[0]
    C = cw.shape[1]
    Bp = max(8, (B + 7) // 8 * 8)

    xb = jnp.pad(x.astype(f32), ((0, Bp - B), (0, 0), (0, 0)))
    x_bcl = jnp.transpose(xb, (0, 2, 1))

    Kmax = max(CONV_KS)
    K2 = Kmax + POOL - 1
    pad_big = (Kmax - 1) // 2
    xpd = jnp.pad(x_bcl, ((0, 0), (0, 0), (pad_big, Kmax - 1 - pad_big)))
    idx = POOL * jnp.arange(T)[:, None] + jnp.arange(K2)[None, :]
    pt = xpd[:, :, idx]
    spatch = jnp.transpose(pt, (2, 0, 1, 3)).reshape(T * Bp, Cin * K2)
    cwp = []
    for p in range(POOL):
        w_p = jnp.zeros((Cin, K2, C), f32)
        r0 = 0
        for K in CONV_KS:
            off = pad_big - (K - 1) // 2
            cwb = cw[r0:r0 + Cin * K].reshape(Cin, K, C)
            w_p = w_p + jnp.pad(
                cwb, ((0, 0), (p + off, K2 - K - p - off), (0, 0)))
            r0 += Cin * K
        cwp.append(w_p.reshape(Cin * K2, C))
    CK2 = Cin * K2

    out = pl.pallas_call(
        functools.partial(_fused_kernel, Bp=Bp, H=H, FCH=FCH),
        out_shape=jax.ShapeDtypeStruct((Bp, 1), f32),
        in_specs=[
            _full((T * Bp, CK2)),
            _full((CK2, C)), _full((CK2, C)), _full((CK2, C)), _full((1, C)),
            _full((C, 8 * H)), _full((1, 8 * H)),
            _full((H, 4 * H)), _full((H, 4 * H)),
            _full((H, 8 * H)), _full((H, 8 * H)), _full((1, 8 * H)),
            _full((H, 4 * H)), _full((H, 4 * H)),
            pl.BlockSpec(memory_space=pl.ANY),
            pl.BlockSpec(memory_space=pl.ANY),
            _full((1, FCH)),
            _full((FCH, FCH)), _full((1, FCH)),
            _full((1, FCH)), _full((1, 1)),
        ],
        out_specs=_full((Bp, 1)),
        scratch_shapes=[
            pltpu.VMEM((T * Bp, 256), f32),
            pltpu.VMEM((256, 256), f32),
            pltpu.VMEM((T * Bp, C), f32),
            pltpu.VMEM((T * Bp, 8 * H), f32),
            pltpu.VMEM((2, H, 4 * H), bf16),
            pltpu.VMEM((2, T * Bp, H), f32),
            pltpu.VMEM((Bp, T * H), f32),
            pltpu.VMEM((Bp, T * H), f32),
            pltpu.VMEM((T * H, FCH), f32),
            pltpu.VMEM((T * H, FCH), f32),
            pltpu.SemaphoreType.DMA,
            pltpu.SemaphoreType.DMA,
        ],
        grid=(),
    )(spatch, cwp[0], cwp[1], cwp[2], cb, wih1, b1, whh1f, whh1b,
      wih2f, wih2b, b2, whh2f, whh2b, fc1wf, fc1wb, fc1b,
      fc2w, fc2b, fc3w, fc3b)

    return out[:B, 0]


# speedup vs baseline: 2.6845x; 1.1066x over previous
import functools

import jax
import jax.numpy as jnp
from jax import lax
from jax.experimental import pallas as pl
from jax.experimental.pallas import tpu as pltpu

T = 36
POOL = 3
CONV_KS = (10, 15)
MC = 144
bf16 = jnp.bfloat16


def _sigmoid(x):
    return 0.5 * (jnp.tanh(0.5 * x) + 1.0)


def _full(shape):
    nd = len(shape)
    return pl.BlockSpec(tuple(shape), lambda _n=nd: (0,) * _n)


def _mm576(lhs_ref, col0, mxu, lsr):
    for j, mc in enumerate(range(0, T * 16, MC)):
        chunk = lhs_ref[pl.ds(mc, MC), pl.ds(col0, 256)].astype(bf16)
        pltpu.matmul_acc_lhs(mc // 4, chunk, mxu,
                             load_staged_rhs=lsr if j == 0 else None)


def _pop576(out_ref, col0, mxu, bias):
    for mc in range(0, T * 16, MC):
        v = pltpu.matmul_pop(mc // 4, (MC, 256), jnp.float32, mxu)
        out_ref[pl.ds(mc, MC), pl.ds(col0, 256)] = v + bias


def _cell(g0, g1, g2, g3, c_prev):
    i = _sigmoid(g0)
    f = _sigmoid(g1)
    g = jnp.tanh(g2)
    o = _sigmoid(g3)
    c = f * c_prev + i * g
    return o * jnp.tanh(c), c


def _lstm_bidir_loop(xp_scr, whh16_scr, store_fwd, store_bwd, Bp, H):
    f32 = jnp.float32
    z = jnp.zeros((Bp, H), f32)
    z16 = jnp.zeros((Bp, H), bf16)

    def push(dirn, tile, msr, mxu):
        pltpu.matmul_push_rhs(
            whh16_scr[dirn, :, pl.ds((2 * mxu + tile) * 256, 256)], msr, mxu)

    for mxu in range(2):
        push(0, 0, 0, mxu)
        pltpu.matmul_acc_lhs(0, z16, mxu, load_staged_rhs=0)

    def gates_f(rf):
        xpf = xp_scr[pl.ds(rf, Bp), :]
        return (pltpu.matmul_pop(0, (Bp, 256), f32, 0) + xpf[:, 0:256],
                pltpu.matmul_pop(8, (Bp, 256), f32, 0) + xpf[:, 256:512],
                pltpu.matmul_pop(0, (Bp, 256), f32, 1) + xpf[:, 512:768],
                pltpu.matmul_pop(8, (Bp, 256), f32, 1) + xpf[:, 768:1024])

    def gates_b(rb):
        xpb = xp_scr[pl.ds(rb, Bp), :]
        return (pltpu.matmul_pop(16, (Bp, 256), f32, 0) + xpb[:, 1024:1280],
                pltpu.matmul_pop(24, (Bp, 256), f32, 0) + xpb[:, 1280:1536],
                pltpu.matmul_pop(16, (Bp, 256), f32, 1) + xpb[:, 1536:1792],
                pltpu.matmul_pop(24, (Bp, 256), f32, 1) + xpb[:, 1792:2048])

    def body(p2, carry):
        hf, cf, hb, cb = carry
        s0 = 2 * p2
        rf = pl.multiple_of(s0 * Bp, Bp)
        rb = pl.multiple_of((T - 1 - s0) * Bp, Bp)
        hf16 = hf.astype(bf16)
        hb16 = hb.astype(bf16)
        for mxu in range(2):
            pltpu.matmul_acc_lhs(0, hf16, mxu, load_staged_rhs=None)
            push(0, 1, 1, mxu)
            pltpu.matmul_acc_lhs(8, hf16, mxu, load_staged_rhs=1)
            push(1, 0, 0, mxu)
            pltpu.matmul_acc_lhs(16, hb16, mxu, load_staged_rhs=0)
            push(1, 1, 1, mxu)
            pltpu.matmul_acc_lhs(24, hb16, mxu, load_staged_rhs=1)
        hf, cf = _cell(*gates_f(rf), cf)
        store_fwd(rf, s0, hf)
        hb, cb = _cell(*gates_b(rb), cb)
        store_bwd(rb, T - 1 - s0, hb)
        s1 = s0 + 1
        rf = pl.multiple_of(s1 * Bp, Bp)
        rb = pl.multiple_of((T - 1 - s1) * Bp, Bp)
        hf16 = hf.astype(bf16)
        hb16 = hb.astype(bf16)
        for mxu in range(2):
            pltpu.matmul_acc_lhs(24, hb16, mxu, load_staged_rhs=None)
            push(1, 0, 0, mxu)
            pltpu.matmul_acc_lhs(16, hb16, mxu, load_staged_rhs=0)
            push(0, 1, 1, mxu)
            pltpu.matmul_acc_lhs(8, hf16, mxu, load_staged_rhs=1)
            push(0, 0, 0, mxu)
            pltpu.matmul_acc_lhs(0, hf16, mxu, load_staged_rhs=0)
        hb, cb = _cell(*gates_b(rb), cb)
        store_bwd(rb, T - 1 - s1, hb)
        hf, cf = _cell(*gates_f(rf), cf)
        store_fwd(rf, s1, hf)
        return hf, cf, hb, cb

    lax.fori_loop(0, T // 2, body, (z, z, z, z))


def _fused_kernel(sp_ref, cwp0_ref, cwp1_ref, cwp2_ref, cb_ref,
                  wih1_ref, b1_ref, whh1f_ref, whh1b_ref,
                  wih2f_ref, wih2b_ref, b2_ref, whh2f_ref, whh2b_ref,
                  fc1wf_hbm, fc1wb_hbm, fc1b_ref,
                  fc2w_ref, fc2b_ref, fc3w_ref, fc3b_ref,
                  o_ref,
                  pscr, cwscr, feat_scr, xp_scr, whh16_scr,
                  l1_scr, h2f_scr, h2b_scr, fc1wf_scr, fc1wb_scr,
                  semf, semb, *, Bp, H, FCH):
    f32 = jnp.float32
    CK2 = cwp0_ref.shape[0]

    pltpu.make_async_copy(fc1wf_hbm, fc1wf_scr, semf).start()
    pltpu.make_async_copy(fc1wb_hbm, fc1wb_scr, semb).start()

    pscr[...] = jnp.zeros((T * Bp, 256), f32)
    pscr[:, pl.ds(0, CK2)] = sp_ref[...]
    cwscr[...] = jnp.zeros((256, 256), f32)

    for p, w_ref in enumerate((cwp0_ref, cwp1_ref, cwp2_ref)):
        cwscr[pl.ds(0, CK2), :] = w_ref[...]
        mxu = p % 2
        pltpu.matmul_push_rhs(cwscr[...].astype(bf16), 0, mxu)
        _mm576(pscr, 0, mxu, 0)
        for mc in range(0, T * Bp, MC):
            v = pltpu.matmul_pop(mc // 4, (MC, 256), f32, mxu)
            if p == 0:
                feat_scr[pl.ds(mc, MC), :] = v
            elif p == 1:
                feat_scr[pl.ds(mc, MC), :] = jnp.maximum(
                    feat_scr[pl.ds(mc, MC), :], v)
            else:
                feat_scr[pl.ds(mc, MC), :] = jnp.maximum(
                    jnp.maximum(feat_scr[pl.ds(mc, MC), :], v) + cb_ref[...],
                    0.0)

    for n in range(8):
        mxu = n % 2
        pltpu.matmul_push_rhs(
            wih1_ref[:, pl.ds(n * 256, 256)].astype(bf16), 0, mxu)
        _mm576(feat_scr, 0, mxu, 0)
        _pop576(xp_scr, n * 256, mxu, b1_ref[0, pl.ds(n * 256, 256)][None, :])

    whh16_scr[0] = whh1f_ref[...].astype(bf16)
    whh16_scr[1] = whh1b_ref[...].astype(bf16)

    def store_fwd1(r, t, h):
        l1_scr[0, pl.ds(r, Bp), :] = h

    def store_bwd1(r, t, h):
        l1_scr[1, pl.ds(r, Bp), :] = h

    _lstm_bidir_loop(xp_scr, whh16_scr, store_fwd1, store_bwd1, Bp, H)

    for n in range(8):
        mxu = n % 2
        pltpu.matmul_push_rhs(
            wih2f_ref[:, pl.ds(n * 256, 256)].astype(bf16), 0, mxu)
        pltpu.matmul_push_rhs(
            wih2b_ref[:, pl.ds(n * 256, 256)].astype(bf16), 1, mxu)
        _mm576(l1_scr.at[0], 0, mxu, 0)
        _mm576(l1_scr.at[1], 0, mxu, 1)
        _pop576(xp_scr, n * 256, mxu, b2_ref[0, pl.ds(n * 256, 256)][None, :])

    whh16_scr[0] = whh2f_ref[...].astype(bf16)
    whh16_scr[1] = whh2b_ref[...].astype(bf16)

    def store_fwd(r, t, h):
        h2f_scr[:, pl.ds(pl.multiple_of(t * H, H), H)] = h

    def store_bwd(r, t, h):
        h2b_scr[:, pl.ds(pl.multiple_of(t * H, H), H)] = h

    _lstm_bidir_loop(xp_scr, whh16_scr, store_fwd, store_bwd, Bp, H)

    pltpu.make_async_copy(fc1wf_hbm, fc1wf_scr, semf).wait()
    pltpu.make_async_copy(fc1wb_hbm, fc1wb_scr, semb).wait()
    for kt in range(T):
        msr = kt % 2
        pltpu.matmul_push_rhs(
            fc1wf_scr[pl.ds(kt * 256, 256), :].astype(bf16), msr, 0)
        pltpu.matmul_acc_lhs(0, h2f_scr[:, pl.ds(kt * 256, 256)].astype(bf16),
                             0, load_staged_rhs=msr)
        pltpu.matmul_push_rhs(
            fc1wb_scr[pl.ds(kt * 256, 256), :].astype(bf16), msr, 1)
        pltpu.matmul_acc_lhs(0, h2b_scr[:, pl.ds(kt * 256, 256)].astype(bf16),
                             1, load_staged_rhs=msr)
    acc = (pltpu.matmul_pop(0, (Bp, FCH), f32, 0)
           + pltpu.matmul_pop(0, (Bp, FCH), f32, 1))

    y = jnp.maximum(acc + fc1b_ref[...], 0.0)
    pltpu.matmul_push_rhs(fc2w_ref[...].astype(bf16), 0, 0)
    pltpu.matmul_acc_lhs(0, y.astype(bf16), 0, load_staged_rhs=0)
    y = jnp.maximum(pltpu.matmul_pop(0, (Bp, FCH), f32, 0)
                    + fc2b_ref[...], 0.0)
    o_ref[...] = jnp.sum(y * fc3w_ref[...], axis=1, keepdims=True) + fc3b_ref[...]


def kernel(x, cw, cb, wih1, b1, whh1f, whh1b, wih2f, wih2b, b2, whh2f, whh2b,
           fc1wf, fc1wb, fc1b, fc2w, fc2b, fc3w, fc3b):
    f32 = jnp.float32
    B, L, Cin = x.shape
    H = whh1f.shape[0]
    FCH = fc2w.shape[0]
    C = cw.shape[1]
    Bp = max(8, (B + 7) // 8 * 8)

    xb = jnp.pad(x.astype(f32), ((0, Bp - B), (0, 0), (0, 0)))
    x_bcl = jnp.transpose(xb, (0, 2, 1))

    Kmax = max(CONV_KS)
    K2 = Kmax + POOL - 1
    pad_big = (Kmax - 1) // 2
    xpd = jnp.pad(x_bcl, ((0, 0), (0, 0), (pad_big, Kmax - 1 - pad_big)))
    idx = POOL * jnp.arange(T)[:, None] + jnp.arange(K2)[None, :]
    pt = xpd[:, :, idx]
    spatch = jnp.transpose(pt, (2, 0, 1, 3)).reshape(T * Bp, Cin * K2)
    cwp = []
    for p in range(POOL):
        w_p = jnp.zeros((Cin, K2, C), f32)
        r0 = 0
        for K in CONV_KS:
            off = pad_big - (K - 1) // 2
            cwb = cw[r0:r0 + Cin * K].reshape(Cin, K, C)
            w_p = w_p + jnp.pad(
                cwb, ((0, 0), (p + off, K2 - K - p - off), (0, 0)))
            r0 += Cin * K
        cwp.append(w_p.reshape(Cin * K2, C))
    CK2 = Cin * K2

    out = pl.pallas_call(
        functools.partial(_fused_kernel, Bp=Bp, H=H, FCH=FCH),
        out_shape=jax.ShapeDtypeStruct((Bp, 1), f32),
        in_specs=[
            _full((T * Bp, CK2)),
            _full((CK2, C)), _full((CK2, C)), _full((CK2, C)), _full((1, C)),
            _full((C, 8 * H)), _full((1, 8 * H)),
            _full((H, 4 * H)), _full((H, 4 * H)),
            _full((H, 8 * H)), _full((H, 8 * H)), _full((1, 8 * H)),
            _full((H, 4 * H)), _full((H, 4 * H)),
            pl.BlockSpec(memory_space=pl.ANY),
            pl.BlockSpec(memory_space=pl.ANY),
            _full((1, FCH)),
            _full((FCH, FCH)), _full((1, FCH)),
            _full((1, FCH)), _full((1, 1)),
        ],
        out_specs=_full((Bp, 1)),
        scratch_shapes=[
            pltpu.VMEM((T * Bp, 256), f32),
            pltpu.VMEM((256, 256), f32),
            pltpu.VMEM((T * Bp, C), f32),
            pltpu.VMEM((T * Bp, 8 * H), f32),
            pltpu.VMEM((2, H, 4 * H), bf16),
            pltpu.VMEM((2, T * Bp, H), f32),
            pltpu.VMEM((Bp, T * H), f32),
            pltpu.VMEM((Bp, T * H), f32),
            pltpu.VMEM((T * H, FCH), f32),
            pltpu.VMEM((T * H, FCH), f32),
            pltpu.SemaphoreType.DMA,
            pltpu.SemaphoreType.DMA,
        ],
        grid=(),
    )(spatch, cwp[0], cwp[1], cwp[2], cb, wih1, b1, whh1f, whh1b,
      wih2f, wih2b, b2, whh2f, whh2b, fc1wf, fc1wb, fc1b,
      fc2w, fc2b, fc3w, fc3b)

    return out[:B, 0]


# speedup vs baseline: 2.8770x; 1.0717x over previous
import functools

import jax
import jax.numpy as jnp
from jax import lax
from jax.experimental import pallas as pl
from jax.experimental.pallas import tpu as pltpu

T = 36
POOL = 3
CONV_KS = (10, 15)
MC = 144
bf16 = jnp.bfloat16


def _sigmoid(x):
    return 0.5 * (jnp.tanh(0.5 * x) + 1.0)


def _full(shape):
    nd = len(shape)
    return pl.BlockSpec(tuple(shape), lambda _n=nd: (0,) * _n)


def _mm576(lhs_ref, col0, mxu, lsr):
    for j, mc in enumerate(range(0, T * 16, MC)):
        chunk = lhs_ref[pl.ds(mc, MC), pl.ds(col0, 256)].astype(bf16)
        pltpu.matmul_acc_lhs(mc // 4, chunk, mxu,
                             load_staged_rhs=lsr if j == 0 else None)


def _pop576(out_ref, col0, mxu, bias):
    for mc in range(0, T * 16, MC):
        v = pltpu.matmul_pop(mc // 4, (MC, 256), jnp.float32, mxu)
        out_ref[pl.ds(mc, MC), pl.ds(col0, 256)] = v + bias


def _cell(g0, g1, g2, g3, c_prev):
    i = _sigmoid(g0)
    f = _sigmoid(g1)
    g = jnp.tanh(g2)
    o = _sigmoid(g3)
    c = f * c_prev + i * g
    return o * jnp.tanh(c), c


def _lstm_bidir_loop(xp_scr, whh16_scr, store_fwd, store_bwd, Bp, H):
    f32 = jnp.float32
    z = jnp.zeros((Bp, H), f32)
    z16 = jnp.zeros((Bp, H), bf16)

    def push(dirn, tile, msr, mxu):
        pltpu.matmul_push_rhs(
            whh16_scr[dirn, :, pl.ds((2 * mxu + tile) * 256, 256)], msr, mxu)

    for mxu in range(2):
        push(0, 0, 0, mxu)
        pltpu.matmul_acc_lhs(0, z16, mxu, load_staged_rhs=0)

    def gates_f(rf):
        xpf = xp_scr[pl.ds(rf, Bp), :]
        return (pltpu.matmul_pop(0, (Bp, 256), f32, 0) + xpf[:, 0:256],
                pltpu.matmul_pop(8, (Bp, 256), f32, 0) + xpf[:, 256:512],
                pltpu.matmul_pop(0, (Bp, 256), f32, 1) + xpf[:, 512:768],
                pltpu.matmul_pop(8, (Bp, 256), f32, 1) + xpf[:, 768:1024])

    def gates_b(rb):
        xpb = xp_scr[pl.ds(rb, Bp), :]
        return (pltpu.matmul_pop(16, (Bp, 256), f32, 0) + xpb[:, 1024:1280],
                pltpu.matmul_pop(24, (Bp, 256), f32, 0) + xpb[:, 1280:1536],
                pltpu.matmul_pop(16, (Bp, 256), f32, 1) + xpb[:, 1536:1792],
                pltpu.matmul_pop(24, (Bp, 256), f32, 1) + xpb[:, 1792:2048])

    def body(p2, carry):
        hf, cf, hb, cb = carry
        s0 = 2 * p2
        rf = pl.multiple_of(s0 * Bp, Bp)
        rb = pl.multiple_of((T - 1 - s0) * Bp, Bp)
        hf16 = hf.astype(bf16)
        hb16 = hb.astype(bf16)
        for mxu in range(2):
            pltpu.matmul_acc_lhs(0, hf16, mxu, load_staged_rhs=None)
            push(0, 1, 1, mxu)
            pltpu.matmul_acc_lhs(8, hf16, mxu, load_staged_rhs=1)
            push(1, 0, 0, mxu)
            pltpu.matmul_acc_lhs(16, hb16, mxu, load_staged_rhs=0)
            push(1, 1, 1, mxu)
            pltpu.matmul_acc_lhs(24, hb16, mxu, load_staged_rhs=1)
        hf, cf = _cell(*gates_f(rf), cf)
        store_fwd(rf, s0, hf)
        hb, cb = _cell(*gates_b(rb), cb)
        store_bwd(rb, T - 1 - s0, hb)
        s1 = s0 + 1
        rf = pl.multiple_of(s1 * Bp, Bp)
        rb = pl.multiple_of((T - 1 - s1) * Bp, Bp)
        hf16 = hf.astype(bf16)
        hb16 = hb.astype(bf16)
        for mxu in range(2):
            pltpu.matmul_acc_lhs(24, hb16, mxu, load_staged_rhs=None)
            push(1, 0, 0, mxu)
            pltpu.matmul_acc_lhs(16, hb16, mxu, load_staged_rhs=0)
            push(0, 1, 1, mxu)
            pltpu.matmul_acc_lhs(8, hf16, mxu, load_staged_rhs=1)
            push(0, 0, 0, mxu)
            pltpu.matmul_acc_lhs(0, hf16, mxu, load_staged_rhs=0)
        hb, cb = _cell(*gates_b(rb), cb)
        store_bwd(rb, T - 1 - s1, hb)
        hf, cf = _cell(*gates_f(rf), cf)
        store_fwd(rf, s1, hf)
        return hf, cf, hb, cb

    lax.fori_loop(0, T // 2, body, (z, z, z, z))


def _fused_kernel(sp_ref, cw_ref, cb_ref,
                  wih1_ref, b1_ref, whh1f_ref, whh1b_ref,
                  wih2f_ref, wih2b_ref, b2_ref, whh2f_ref, whh2b_ref,
                  fc1wf_hbm, fc1wb_hbm, fc1b_ref,
                  fc2w_ref, fc2b_ref, fc3w_ref, fc3b_ref,
                  o_ref,
                  pscr, cwscr, feat_scr, xp_scr, whh16_scr,
                  l1_scr, h2f_scr, h2b_scr, fc1wf_scr, fc1wb_scr,
                  semf, semb, *, Bp, H, FCH):
    f32 = jnp.float32
    CK2 = sp_ref.shape[1]
    K2 = CK2 // 4

    pltpu.make_async_copy(fc1wf_hbm, fc1wf_scr, semf).start()
    pltpu.make_async_copy(fc1wb_hbm, fc1wb_scr, semb).start()

    pscr[...] = jnp.zeros((T * Bp, 256), bf16)
    pscr[:, pl.ds(0, CK2)] = sp_ref[...]

    for p in range(POOL):
        cwscr[...] = jnp.zeros((256, 256), f32)
        r0 = 0
        for K in CONV_KS:
            off = p + (CONV_KS[-1] - 1) // 2 - (K - 1) // 2
            for c in range(4):
                cwscr[pl.ds(c * K2 + off, K), :] = (
                    cw_ref[pl.ds(r0 + c * K, K), :])
            r0 += 4 * K
        mxu = p % 2
        pltpu.matmul_push_rhs(cwscr[...].astype(bf16), 0, mxu)
        _mm576(pscr, 0, mxu, 0)
        for mc in range(0, T * Bp, MC):
            v = pltpu.matmul_pop(mc // 4, (MC, 256), f32, mxu)
            if p == 0:
                feat_scr[pl.ds(mc, MC), :] = v
            elif p == 1:
                feat_scr[pl.ds(mc, MC), :] = jnp.maximum(
                    feat_scr[pl.ds(mc, MC), :], v)
            else:
                feat_scr[pl.ds(mc, MC), :] = jnp.maximum(
                    jnp.maximum(feat_scr[pl.ds(mc, MC), :], v) + cb_ref[...],
                    0.0)

    for n in range(8):
        mxu = n % 2
        pltpu.matmul_push_rhs(
            wih1_ref[:, pl.ds(n * 256, 256)].astype(bf16), 0, mxu)
        _mm576(feat_scr, 0, mxu, 0)
        _pop576(xp_scr, n * 256, mxu, b1_ref[0, pl.ds(n * 256, 256)][None, :])

    whh16_scr[0] = whh1f_ref[...].astype(bf16)
    whh16_scr[1] = whh1b_ref[...].astype(bf16)

    def store_fwd1(r, t, h):
        l1_scr[0, pl.ds(r, Bp), :] = h

    def store_bwd1(r, t, h):
        l1_scr[1, pl.ds(r, Bp), :] = h

    _lstm_bidir_loop(xp_scr, whh16_scr, store_fwd1, store_bwd1, Bp, H)

    for n in range(8):
        mxu = n % 2
        pltpu.matmul_push_rhs(
            wih2f_ref[:, pl.ds(n * 256, 256)].astype(bf16), 0, mxu)
        pltpu.matmul_push_rhs(
            wih2b_ref[:, pl.ds(n * 256, 256)].astype(bf16), 1, mxu)
        _mm576(l1_scr.at[0], 0, mxu, 0)
        _mm576(l1_scr.at[1], 0, mxu, 1)
        _pop576(xp_scr, n * 256, mxu, b2_ref[0, pl.ds(n * 256, 256)][None, :])

    whh16_scr[0] = whh2f_ref[...].astype(bf16)
    whh16_scr[1] = whh2b_ref[...].astype(bf16)

    def store_fwd(r, t, h):
        h2f_scr[:, pl.ds(pl.multiple_of(t * H, H), H)] = h

    def store_bwd(r, t, h):
        h2b_scr[:, pl.ds(pl.multiple_of(t * H, H), H)] = h

    _lstm_bidir_loop(xp_scr, whh16_scr, store_fwd, store_bwd, Bp, H)

    pltpu.make_async_copy(fc1wf_hbm, fc1wf_scr, semf).wait()
    pltpu.make_async_copy(fc1wb_hbm, fc1wb_scr, semb).wait()
    for kt in range(T):
        msr = kt % 2
        pltpu.matmul_push_rhs(
            fc1wf_scr[pl.ds(kt * 256, 256), :].astype(bf16), msr, 0)
        pltpu.matmul_acc_lhs(0, h2f_scr[:, pl.ds(kt * 256, 256)].astype(bf16),
                             0, load_staged_rhs=msr)
        pltpu.matmul_push_rhs(
            fc1wb_scr[pl.ds(kt * 256, 256), :].astype(bf16), msr, 1)
        pltpu.matmul_acc_lhs(0, h2b_scr[:, pl.ds(kt * 256, 256)].astype(bf16),
                             1, load_staged_rhs=msr)
    acc = (pltpu.matmul_pop(0, (Bp, FCH), f32, 0)
           + pltpu.matmul_pop(0, (Bp, FCH), f32, 1))

    y = jnp.maximum(acc + fc1b_ref[...], 0.0)
    pltpu.matmul_push_rhs(fc2w_ref[...].astype(bf16), 0, 0)
    pltpu.matmul_acc_lhs(0, y.astype(bf16), 0, load_staged_rhs=0)
    y = jnp.maximum(pltpu.matmul_pop(0, (Bp, FCH), f32, 0)
                    + fc2b_ref[...], 0.0)
    o_ref[...] = jnp.sum(y * fc3w_ref[...], axis=1, keepdims=True) + fc3b_ref[...]


def kernel(x, cw, cb, wih1, b1, whh1f, whh1b, wih2f, wih2b, b2, whh2f, whh2b,
           fc1wf, fc1wb, fc1b, fc2w, fc2b, fc3w, fc3b):
    f32 = jnp.float32
    B, L, Cin = x.shape
    H = whh1f.shape[0]
    FCH = fc2w.shape[0]
    C = cw.shape[1]
    Bp = max(8, (B + 7) // 8 * 8)

    xb = jnp.pad(x.astype(f32), ((0, Bp - B), (0, 0), (0, 0)))
    x_bcl = jnp.transpose(xb, (0, 2, 1))

    Kmax = max(CONV_KS)
    K2 = Kmax + POOL - 1
    pad_big = (Kmax - 1) // 2
    xpd = jnp.pad(x_bcl.astype(bf16),
                  ((0, 0), (0, 0), (pad_big, Kmax - 1 - pad_big)))
    idx = POOL * jnp.arange(T)[:, None] + jnp.arange(K2)[None, :]
    pt = xpd[:, :, idx]
    spatch = jnp.transpose(pt, (2, 0, 1, 3)).reshape(T * Bp, Cin * K2)
    CK2 = Cin * K2

    out = pl.pallas_call(
        functools.partial(_fused_kernel, Bp=Bp, H=H, FCH=FCH),
        out_shape=jax.ShapeDtypeStruct((Bp, 1), f32),
        in_specs=[
            _full((T * Bp, CK2)),
            _full((cw.shape[0], C)), _full((1, C)),
            _full((C, 8 * H)), _full((1, 8 * H)),
            _full((H, 4 * H)), _full((H, 4 * H)),
            _full((H, 8 * H)), _full((H, 8 * H)), _full((1, 8 * H)),
            _full((H, 4 * H)), _full((H, 4 * H)),
            pl.BlockSpec(memory_space=pl.ANY),
            pl.BlockSpec(memory_space=pl.ANY),
            _full((1, FCH)),
            _full((FCH, FCH)), _full((1, FCH)),
            _full((1, FCH)), _full((1, 1)),
        ],
        out_specs=_full((Bp, 1)),
        scratch_shapes=[
            pltpu.VMEM((T * Bp, 256), bf16),
            pltpu.VMEM((256, 256), f32),
            pltpu.VMEM((T * Bp, C), f32),
            pltpu.VMEM((T * Bp, 8 * H), f32),
            pltpu.VMEM((2, H, 4 * H), bf16),
            pltpu.VMEM((2, T * Bp, H), f32),
            pltpu.VMEM((Bp, T * H), f32),
            pltpu.VMEM((Bp, T * H), f32),
            pltpu.VMEM((T * H, FCH), f32),
            pltpu.VMEM((T * H, FCH), f32),
            pltpu.SemaphoreType.DMA,
            pltpu.SemaphoreType.DMA,
        ],
        grid=(),
    )(spatch, cw, cb, wih1, b1, whh1f, whh1b,
      wih2f, wih2b, b2, whh2f, whh2b, fc1wf, fc1wb, fc1b,
      fc2w, fc2b, fc3w, fc3b)

    return out[:B, 0]
